# baseline (speedup 1.0000x reference)
import functools
import math

import jax
import jax.numpy as jnp
from jax import lax
from jax.experimental import pallas as pl
from jax.experimental.pallas import tpu as pltpu, tpu_sc as plsc

HID = 96
EMB = 64

N_NODES = 50000
N_EDGES = 800000
BLK_N = 2000
BLK_E = 3200


def _ln_silu(h, g, b):
    m = h.mean(axis=-1, keepdims=True)
    v = ((h - m) ** 2).mean(axis=-1, keepdims=True)
    h = (h - m) / jnp.sqrt(v + 1e-5) * g + b
    return h * jax.nn.sigmoid(h)


def _emb2_kernel(xs_ref, w1_ref, b1_ref, g1_ref, n1_ref, w2_ref, b2_ref,
                 g2_ref, n2_ref, o_ref, *, vmin, vmax, bins):
    xs = xs_ref[...]
    delta = (vmax - vmin) / (bins - 1)
    centers = vmin + delta * lax.broadcasted_iota(jnp.int32, (1, bins), 1).astype(jnp.float32)
    gamma = 1.0 / (delta * delta)
    r = jnp.exp(-gamma * (xs - centers) ** 2)
    h = _ln_silu(r @ w1_ref[...] + b1_ref[...], g1_ref[...], n1_ref[...])
    h = _ln_silu(h @ w2_ref[...] + b2_ref[...], g2_ref[...], n2_ref[...])
    o_ref[...] = h


def _emb2(xs, p1, p2, vmin, vmax, bins, blk):
    rows = xs.shape[0]
    d1 = p1["lin"]["w"].shape[1]
    d2 = p2["lin"]["w"].shape[1]
    f = pl.pallas_call(
        functools.partial(_emb2_kernel, vmin=vmin, vmax=vmax, bins=bins),
        grid=(rows // blk,),
        in_specs=[
            pl.BlockSpec((blk, 1), lambda i: (i, 0)),
            pl.BlockSpec((bins, d1), lambda i: (0, 0)),
            pl.BlockSpec((1, d1), lambda i: (0, 0)),
            pl.BlockSpec((1, d1), lambda i: (0, 0)),
            pl.BlockSpec((1, d1), lambda i: (0, 0)),
            pl.BlockSpec((d1, d2), lambda i: (0, 0)),
            pl.BlockSpec((1, d2), lambda i: (0, 0)),
            pl.BlockSpec((1, d2), lambda i: (0, 0)),
            pl.BlockSpec((1, d2), lambda i: (0, 0)),
        ],
        out_specs=pl.BlockSpec((blk, d2), lambda i: (i, 0)),
        out_shape=jax.ShapeDtypeStruct((rows, d2), jnp.float32),
    )
    r2 = lambda a: a.reshape(1, -1)
    return f(xs[:, None], p1["lin"]["w"], r2(p1["lin"]["b"]), r2(p1["ln"]["g"]),
             r2(p1["ln"]["b"]), p2["lin"]["w"], r2(p2["lin"]["b"]),
             r2(p2["ln"]["g"]), r2(p2["ln"]["b"]))


def _atom_emb_kernel(x_ref, w_ref, b_ref, g_ref, n_ref, o_ref):
    h = x_ref[...] @ w_ref[...] + b_ref[...]
    o_ref[...] = _ln_silu(h, g_ref[...], n_ref[...])


def _atom_emb(x, p):
    rows, din = x.shape
    f = pl.pallas_call(
        _atom_emb_kernel,
        grid=(rows // BLK_N,),
        in_specs=[
            pl.BlockSpec((BLK_N, din), lambda i: (i, 0)),
            pl.BlockSpec((din, HID), lambda i: (0, 0)),
            pl.BlockSpec((1, HID), lambda i: (0, 0)),
            pl.BlockSpec((1, HID), lambda i: (0, 0)),
            pl.BlockSpec((1, HID), lambda i: (0, 0)),
        ],
        out_specs=pl.BlockSpec((BLK_N, HID), lambda i: (i, 0)),
        out_shape=jax.ShapeDtypeStruct((rows, HID), jnp.float32),
    )
    r2 = lambda a: a.reshape(1, -1)
    return f(x, p["lin"]["w"], r2(p["lin"]["b"]), r2(p["ln"]["g"]), r2(p["ln"]["b"]))


def _time_kernel(ts_ref, w1_ref, b1_ref, g1_ref, n1_ref, w2_ref, b2_ref,
                 g2_ref, n2_ref, wp_ref, bp_ref, o_ref):
    ts = ts_ref[...]
    half = EMB // 2
    fr = math.log(10000.0) / (half - 1)
    freqs = jnp.exp(lax.broadcasted_iota(jnp.int32, (1, half), 1).astype(jnp.float32) * -fr)
    a = ts * freqs
    t = jnp.concatenate([jnp.sin(a), jnp.cos(a)], axis=1)
    t = _ln_silu(t @ w1_ref[...] + b1_ref[...], g1_ref[...], n1_ref[...])
    t = _ln_silu(t @ w2_ref[...] + b2_ref[...], g2_ref[...], n2_ref[...])
    o_ref[...] = t @ wp_ref[...] + bp_ref[...]


def _time_tp(timesteps, params, n_layers_tp, wp_all, bp_all):
    p1, p2 = params["time_emb"]
    ts8 = jnp.zeros((8, 1), jnp.float32).at[0, 0].set(timesteps[0])
    r2 = lambda a: a.reshape(1, -1)
    f = pl.pallas_call(
        _time_kernel,
        out_shape=jax.ShapeDtypeStruct((8, n_layers_tp * HID), jnp.float32),
    )
    out = f(ts8, p1["lin"]["w"], r2(p1["lin"]["b"]), r2(p1["ln"]["g"]), r2(p1["ln"]["b"]),
            p2["lin"]["w"], r2(p2["lin"]["b"]), r2(p2["ln"]["g"]), r2(p2["ln"]["b"]),
            wp_all, r2(bp_all))
    return out[0].reshape(n_layers_tp, HID)


def _pre_kernel(x_ref, w_ref, b_ref, esrc_ref, edst_ref, bh_ref, xu_ref):
    r = x_ref[...] @ w_ref[...] + b_ref[...]
    blk = r.shape[0]
    z = jnp.zeros((blk, 128 - HID), jnp.float32)
    esrc_ref[...] = jnp.concatenate([r[:, 0:96], z], axis=1)
    edst_ref[...] = jnp.concatenate([r[:, 96:192], z], axis=1)
    bh_ref[...] = jnp.concatenate([r[:, 192:288], z], axis=1)
    xu_ref[...] = r[:, 288:384]


def _egc_pre(x, wcat, bcat, blk):
    rows = x.shape[0]
    f = pl.pallas_call(
        _pre_kernel,
        grid=(rows // blk,),
        in_specs=[
            pl.BlockSpec((blk, HID), lambda i: (i, 0)),
            pl.BlockSpec((HID, 384), lambda i: (0, 0)),
            pl.BlockSpec((1, 384), lambda i: (0, 0)),
        ],
        out_specs=[
            pl.BlockSpec((blk, 128), lambda i: (i, 0)),
            pl.BlockSpec((blk, 128), lambda i: (i, 0)),
            pl.BlockSpec((blk, 128), lambda i: (i, 0)),
            pl.BlockSpec((blk, HID), lambda i: (i, 0)),
        ],
        out_shape=[
            jax.ShapeDtypeStruct((rows, 128), jnp.float32),
            jax.ShapeDtypeStruct((rows, 128), jnp.float32),
            jax.ShapeDtypeStruct((rows, 128), jnp.float32),
            jax.ShapeDtypeStruct((rows, HID), jnp.float32),
        ],
    )
    return f(x, wcat, bcat.reshape(1, -1))


def _ge_kernel(y_ref, w_ref, b_ref, o_ref):
    r = y_ref[...] @ w_ref[...] + b_ref[...]
    blk = r.shape[0]
    z = jnp.zeros((blk, 128 - HID), jnp.float32)
    o_ref[...] = jnp.concatenate([r, z], axis=1)


def _egc_ge(y, w, b, blk, out_rows):
    rows = y.shape[0]
    f = pl.pallas_call(
        _ge_kernel,
        grid=(rows // blk,),
        in_specs=[
            pl.BlockSpec((blk, HID), lambda i: (i, 0)),
            pl.BlockSpec((HID, HID), lambda i: (0, 0)),
            pl.BlockSpec((1, HID), lambda i: (0, 0)),
        ],
        out_specs=pl.BlockSpec((blk, 128), lambda i: (i, 0)),
        out_shape=jax.ShapeDtypeStruct((out_rows, 128), jnp.float32),
    )
    return f(y, w, b.reshape(1, -1))


def _post_x_kernel(x_ref, xu_ref, acc_ref, g_ref, b_ref, o_ref):
    acc = acc_ref[...]
    h = acc[:, 0:96] / (acc[:, 96:192] + 1e-6)
    xo = _ln_silu(xu_ref[...] + h, g_ref[...], b_ref[...])
    o_ref[...] = x_ref[...] + xo


def _egc_post_x(x, xu, acc, g, b, blk):
    rows = x.shape[0]
    f = pl.pallas_call(
        _post_x_kernel,
        grid=(rows // blk,),
        in_specs=[
            pl.BlockSpec((blk, HID), lambda i: (i, 0)),
            pl.BlockSpec((blk, HID), lambda i: (i, 0)),
            pl.BlockSpec((blk, 192), lambda i: (i, 0)),
            pl.BlockSpec((1, HID), lambda i: (0, 0)),
            pl.BlockSpec((1, HID), lambda i: (0, 0)),
        ],
        out_specs=pl.BlockSpec((blk, HID), lambda i: (i, 0)),
        out_shape=jax.ShapeDtypeStruct((rows, HID), jnp.float32),
    )
    return f(x, xu, acc, g.reshape(1, -1), b.reshape(1, -1))


def _post_y_kernel(y_ref, m_ref, g_ref, b_ref, o_ref):
    yo = _ln_silu(m_ref[...][:, 0:96], g_ref[...], b_ref[...])
    o_ref[...] = y_ref[...] + yo


def _egc_post_y(y, m_arr, g, b, blk):
    rows = y.shape[0]
    f = pl.pallas_call(
        _post_y_kernel,
        grid=(rows // blk,),
        in_specs=[
            pl.BlockSpec((blk, HID), lambda i: (i, 0)),
            pl.BlockSpec((blk, 128), lambda i: (i, 0)),
            pl.BlockSpec((1, HID), lambda i: (0, 0)),
            pl.BlockSpec((1, HID), lambda i: (0, 0)),
        ],
        out_specs=pl.BlockSpec((blk, HID), lambda i: (i, 0)),
        out_shape=jax.ShapeDtypeStruct((rows, HID), jnp.float32),
    )
    return f(y, m_arr, g.reshape(1, -1), b.reshape(1, -1))


def _readout_kernel(x_ref, w_ref, b_ref, o_ref):
    o_ref[...] = x_ref[...] @ w_ref[...] + b_ref[...]


def _readout(p, x, blk):
    rows = x.shape[0]
    f = pl.pallas_call(
        _readout_kernel,
        grid=(rows // blk,),
        in_specs=[
            pl.BlockSpec((blk, HID), lambda i: (i, 0)),
            pl.BlockSpec((HID, 1), lambda i: (0, 0)),
            pl.BlockSpec((1, 1), lambda i: (0, 0)),
        ],
        out_specs=pl.BlockSpec((blk, 1), lambda i: (i, 0)),
        out_shape=jax.ShapeDtypeStruct((rows, 1), jnp.float32),
    )
    return f(x, p["w"], p["b"].reshape(1, 1))


NC, NS, L = 2, 16, 16
NW = NC * NS
EG = N_EDGES
PCH = 2048
NCHUNKS = (EG + PCH - 1) // PCH
LAST_N = EG - (NCHUNKS - 1) * PCH
KC = 64

GP_NODE = (128, 7, 416, 13, 416 * 128)
GP_EDGE = (256, 8, 3136, 98, 3136 * 256)
NB_HALF_MAX = 1568
TRASH = EG + PCH

_MESH = plsc.VectorSubcoreMesh(core_axis_name="c", subcore_axis_name="s")


def _wid():
    return lax.axis_index("s") * NC + lax.axis_index("c")


def _lane_iota():
    return lax.iota(jnp.int32, L)


def _sel_lane(vec, k):
    sc = vec[0]
    for l in range(1, L):
        sc = jnp.where(k == l, vec[l], sc)
    return sc


def _hist_body(shift, nb, hb, dst_hbm, cnt_hbm, dst_v, cnt_v, hist_s):
    w = _wid()

    def _z(i, _):
        hist_s[i] = 0
        return 0
    lax.fori_loop(0, nb, _z, 0)

    nrounds = (NCHUNKS - w + NW - 1) // NW

    def _round(k, _):
        c = w + k * NW
        cs = c * PCH

        @pl.when(c < NCHUNKS - 1)
        def _():
            pltpu.sync_copy(dst_hbm.at[pl.ds(cs, PCH)], dst_v)

        @pl.when(c == NCHUNKS - 1)
        def _():
            pltpu.sync_copy(dst_hbm.at[pl.ds(cs, LAST_N)], dst_v.at[pl.ds(0, LAST_N)])

        ng = jnp.where(c == NCHUNKS - 1, LAST_N // L, PCH // L)

        def _grp(g, _):
            b16 = lax.shift_right_logical(dst_v[pl.ds(g * L, L)], shift) - hb
            for l in range(L):
                b = b16[l]

                @pl.when((b >= 0) & (b < nb))
                def _():
                    hist_s[b] = hist_s[b] + 1
            return 0
        lax.fori_loop(0, ng, _grp, 0)
        return 0
    lax.fori_loop(0, nrounds, _round, 0)

    def _flush(g, _):
        v = jnp.zeros((L,), jnp.int32)
        io = _lane_iota()
        for l in range(L):
            v = jnp.where(io == l, hist_s[g * L + l], v)
        cnt_v[pl.ds(g * L, L)] = v
        return 0
    lax.fori_loop(0, nb // L, _flush, 0)
    pltpu.sync_copy(cnt_v, cnt_hbm.at[w])


def _prep_hist(dst, gp, hb, nb):
    _, shift, _, _, _ = gp
    f = pl.kernel(
        functools.partial(_hist_body, shift, nb, hb),
        out_type=[jax.ShapeDtypeStruct((NW, nb), jnp.int32)],
        mesh=_MESH,
        name="prep_hist",
        scratch_types=[
            pltpu.VMEM((PCH,), jnp.int32),
            pltpu.VMEM((nb,), jnp.int32),
            pltpu.SMEM((nb,), jnp.int32),
        ],
    )
    return f(dst)[0]


def _scat_body(shift, nb, hb, dst_hbm, src_hbm, cnt_hbm, base_hbm,
               perm_hbm, srcp_hbm, dstp_hbm, bst_hbm,
               cnt_v, dst_v, src_v, bst_v, base_v,
               pos_b, id_b, src_b, dst_b, off_s, sem):
    w = _wid()
    io = _lane_iota()
    pltpu.sync_copy(cnt_hbm, cnt_v)
    pltpu.sync_copy(base_hbm, base_v)
    base0 = base_v[pl.ds(0, L)][0]

    def _off(b, base):
        cv0 = cnt_v[pl.ds(b * NW, L)]
        cv1 = cnt_v[pl.ds(b * NW + L, L)]
        excl = jnp.int32(0)
        tot = jnp.int32(0)
        for l in range(L):
            el = cv0[l]
            excl = excl + jnp.where(w > l, el, 0)
            tot = tot + el
        for l in range(L):
            el = cv1[l]
            excl = excl + jnp.where(w > L + l, el, 0)
            tot = tot + el
        off_s[b] = base + excl
        return base + tot
    lax.fori_loop(0, nb, _off, base0)

    @pl.when(w == 0)
    def _():
        def _fl(g, _):
            v = jnp.zeros((L,), jnp.int32)
            for l in range(L):
                v = jnp.where(io == l, off_s[g * L + l], v)
            bst_v[pl.ds(g * L, L)] = v
            return 0
        lax.fori_loop(0, nb // L, _fl, 0)
        pltpu.sync_copy(bst_v, bst_hbm)

    nrounds = (NCHUNKS - w + NW - 1) // NW

    def _round(k, _):
        c = w + k * NW
        cs = c * PCH

        @pl.when(c < NCHUNKS - 1)
        def _():
            pltpu.sync_copy(dst_hbm.at[pl.ds(cs, PCH)], dst_v)
            pltpu.sync_copy(src_hbm.at[pl.ds(cs, PCH)], src_v)

        @pl.when(c == NCHUNKS - 1)
        def _():
            pltpu.sync_copy(dst_hbm.at[pl.ds(cs, LAST_N)], dst_v.at[pl.ds(0, LAST_N)])
            pltpu.sync_copy(src_hbm.at[pl.ds(cs, LAST_N)], src_v.at[pl.ds(0, LAST_N)])

        ngg = jnp.where(c == NCHUNKS - 1, LAST_N // (8 * L), PCH // (8 * L))

        def _row(gg, _):
            for q in range(8):
                o = gg * 8 * L + q * L
                d16 = dst_v[pl.ds(o, L)]
                s16 = src_v[pl.ds(o, L)]
                b16 = lax.shift_right_logical(d16, shift) - hb
                id16 = io + (cs + o)
                posv = TRASH + o + io
                for l in range(L):
                    b = b16[l]
                    inh = (b >= 0) & (b < nb)
                    bc = jnp.clip(b, 0, nb - 1)

                    @pl.when(inh)
                    def _():
                        off_s[bc] = off_s[bc] + 1

                    p2 = jnp.where(inh, off_s[bc] - 1, TRASH + o + l)
                    posv = jnp.where(io == l, p2, posv)
                pos_b[gg, pl.ds(q * L, L)] = posv
                id_b[gg, pl.ds(q * L, L)] = id16
                src_b[gg, pl.ds(q * L, L)] = s16
                dst_b[gg, pl.ds(q * L, L)] = d16
            return 0
        lax.fori_loop(0, ngg, _row, 0)

        def _scat_row(j, _):
            pltpu.async_copy(id_b.at[j], perm_hbm.at[pos_b.at[j]], sem).wait()
            pltpu.async_copy(src_b.at[j], srcp_hbm.at[pos_b.at[j]], sem).wait()
            pltpu.async_copy(dst_b.at[j], dstp_hbm.at[pos_b.at[j]], sem).wait()
            return 0
        lax.fori_loop(0, ngg, _scat_row, 0)
        return 0
    lax.fori_loop(0, nrounds, _round, 0)

    @pl.when(w == NW - 1)
    def _():
        def _pv(g, _):
            dst_v[pl.ds(g * L, L)] = jnp.full((L,), EG, jnp.int32)
            src_v[pl.ds(g * L, L)] = jnp.zeros((L,), jnp.int32)
            return 0
        lax.fori_loop(0, PCH // L, _pv, 0)
        pltpu.sync_copy(dst_v, perm_hbm.at[pl.ds(EG, PCH)])
        pltpu.sync_copy(src_v, srcp_hbm.at[pl.ds(EG, PCH)])
        pltpu.sync_copy(src_v, dstp_hbm.at[pl.ds(EG, PCH)])


def _prep_scatter(dst, src, cnt_t, base0, gp, hb, nb):
    _, shift, _, _, _ = gp
    f = pl.kernel(
        functools.partial(_scat_body, shift, nb, hb),
        out_type=[jax.ShapeDtypeStruct((EG + 2 * PCH + 2048,), jnp.int32),
                  jax.ShapeDtypeStruct((EG + 2 * PCH + 2048,), jnp.int32),
                  jax.ShapeDtypeStruct((EG + 2 * PCH + 2048,), jnp.int32),
                  jax.ShapeDtypeStruct((nb,), jnp.int32)],
        mesh=_MESH,
        name="prep_scat",
        scratch_types=[
            pltpu.VMEM((nb * NW,), jnp.int32),
            pltpu.VMEM((PCH,), jnp.int32),
            pltpu.VMEM((PCH,), jnp.int32),
            pltpu.VMEM((nb,), jnp.int32),
            pltpu.VMEM((L,), jnp.int32),
            pltpu.VMEM((PCH // (8 * L), 8 * L), jnp.int32),
            pltpu.VMEM((PCH // (8 * L), 8 * L), jnp.int32),
            pltpu.VMEM((PCH // (8 * L), 8 * L), jnp.int32),
            pltpu.VMEM((PCH // (8 * L), 8 * L), jnp.int32),
            pltpu.SMEM((nb,), jnp.int32),
            pltpu.SemaphoreType.DMA,
        ],
    )
    return f(dst, src, cnt_t.reshape(-1), base0)


def _graph_prep(src, dst, gp):
    _, _, b_pad, _, _ = gp
    zero16 = jnp.zeros((L,), jnp.int32)
    if b_pad <= NB_HALF_MAX:
        cnt = _prep_hist(dst, gp, 0, b_pad)
        perm, srcp, dstp, bst = _prep_scatter(
            dst, src, jnp.transpose(cnt), zero16, gp, 0, b_pad)
        bstart = jnp.concatenate([bst, jnp.full((L,), EG, jnp.int32)])
        return perm, srcp, dstp, bstart
    nb = b_pad // 2
    cnt1 = _prep_hist(dst, gp, 0, nb)
    cnt2 = _prep_hist(dst, gp, nb, nb)
    n1 = jnp.sum(cnt1).astype(jnp.int32)
    base2 = zero16.at[0].set(n1)
    p1, s1, d1, bst1 = _prep_scatter(dst, src, jnp.transpose(cnt1), zero16, gp, 0, nb)
    p2, s2, d2, bst2 = _prep_scatter(dst, src, jnp.transpose(cnt2), base2, gp, nb, nb)
    pos = lax.broadcasted_iota(jnp.int32, p1.shape, 0)
    first = pos < n1
    perm = jnp.where(first, p1, p2)
    srcp = jnp.where(first, s1, s2)
    dstp = jnp.where(first, d1, d2)
    bstart = jnp.concatenate([bst1, bst2, jnp.full((L,), EG, jnp.int32)])
    return perm, srcp, dstp, bstart


def _egc_sc_body(w_bucket, b_pad, bpw, ge_rows,
                 esrc_hbm, edst_hbm, bh_hbm, ge_hbm,
                 perm_hbm, srcp_hbm, dstp_hbm, bst_hbm,
                 m_hbm, acc_hbm,
                 pc, sc_, dc, m_buf, bh_buf, acc, bst16, sema, semb, semc):
    w = _wid()
    io = _lane_iota()
    zf = jnp.zeros((L,), jnp.float32)

    def _bucket(bi, _):
        b = w * bpw + bi
        al = (b // 8) * 8
        pltpu.sync_copy(bst_hbm.at[pl.ds(al, L)], bst16)
        bv = bst16[pl.ds(0, L)]
        e0 = _sel_lane(bv, b - al)
        e1 = _sel_lane(bv, b - al + 1)

        def _zr(r, _):
            for j in range(12):
                acc[r, pl.ds(j * L, L)] = zf
            return 0
        lax.fori_loop(0, w_bucket, _zr, 0)

        cs0 = (e0 // 8) * 8
        nch = (e1 - cs0 + KC - 1) // KC

        def _chunk(ci, _):
            cs = cs0 + ci * KC
            lo = jnp.maximum(e0 - cs, 0)
            hi = jnp.minimum(e1 - cs, KC)
            pltpu.sync_copy(perm_hbm.at[pl.ds(cs, KC)], pc)
            pltpu.sync_copy(srcp_hbm.at[pl.ds(cs, KC)], sc_)
            pltpu.sync_copy(dstp_hbm.at[pl.ds(cs, KC)], dc)
            pltpu.async_copy(ge_hbm.at[pc], m_buf, sema).wait()
            d1 = pltpu.async_copy(esrc_hbm.at[sc_], m_buf, sema, add=True)
            d2 = pltpu.async_copy(edst_hbm.at[dc], m_buf, sema, add=True)
            d3 = pltpu.async_copy(bh_hbm.at[sc_], bh_buf, semb)
            d1.wait()
            d2.wait()
            d3.wait()
            bw = b * w_bucket
            for q in range(KC // L):
                dls = dc[pl.ds(q * L, L)] - bw
                for l in range(L):
                    i = q * L + l
                    dl = dls[l]

                    @pl.when((i >= lo) & (i < hi))
                    def _():
                        for j in range(6):
                            m = m_buf[i, pl.ds(j * L, L)]
                            s = 1.0 / (1.0 + jnp.exp(-m))
                            bhv = bh_buf[i, pl.ds(j * L, L)]
                            plsc.addupdate(acc.at[dl, pl.ds(j * L, L)], s * bhv)
                            plsc.addupdate(acc.at[dl, pl.ds(96 + j * L, L)], s)
            pltpu.async_copy(m_buf, m_hbm.at[pc], semc).wait()
            return 0
        lax.fori_loop(0, nch, _chunk, 0)
        pltpu.sync_copy(acc, acc_hbm.at[pl.ds(b * w_bucket, w_bucket)])
        return 0
    lax.fori_loop(0, bpw, _bucket, 0)


def _egc_edge_sc(esrc, edst, bh, ge, prep, gp):
    perm, srcp, dstp, bst = prep
    w_bucket, _, b_pad, bpw, s_pad = gp
    f = pl.kernel(
        functools.partial(_egc_sc_body, w_bucket, b_pad, bpw, ge.shape[0]),
        out_type=[jax.ShapeDtypeStruct((EG + PCH, 128), jnp.float32),
                  jax.ShapeDtypeStruct((s_pad, 192), jnp.float32)],
        mesh=_MESH,
        name="egc_edge",
        scratch_types=[
            pltpu.VMEM((KC,), jnp.int32),
            pltpu.VMEM((KC,), jnp.int32),
            pltpu.VMEM((KC,), jnp.int32),
            pltpu.VMEM((KC, 128), jnp.float32),
            pltpu.VMEM((KC, 128), jnp.float32),
            pltpu.VMEM((w_bucket, 192), jnp.float32),
            pltpu.VMEM((L,), jnp.int32),
            pltpu.SemaphoreType.DMA,
            pltpu.SemaphoreType.DMA,
            pltpu.SemaphoreType.DMA,
        ],
    )
    return f(esrc, edst, bh, ge, perm, srcp, dstp, bst)


def _edge_phase(esrc, edst, bh, ge, prep, gp):
    m_arr, acc = _egc_edge_sc(esrc, edst, bh, ge, prep, gp)
    return m_arr, acc


def _egc_layer(p, tp, prep, gp, x, y, blk_x, blk_y):
    wcat = jnp.concatenate([p["src_gate"]["w"], p["dst_gate"]["w"],
                            p["dst_update"]["w"], p["src_update"]["w"]], axis=1)
    bcat = jnp.concatenate([p["src_gate"]["b"] + tp, p["dst_gate"]["b"],
                            p["dst_update"]["b"], p["src_update"]["b"]])
    esrc, edst, bh, xu = _egc_pre(x, wcat, bcat, blk_x)
    ge = _egc_ge(y, p["edge_gate"]["w"], p["edge_gate"]["b"], blk_y, EG + 8)
    m_arr, acc = _edge_phase(esrc, edst, bh, ge, prep, gp)
    x_new = _egc_post_x(x, xu, acc, p["ln_n"]["g"], p["ln_n"]["b"], blk_x)
    y_new = _egc_post_y(y, m_arr, p["ln_e"]["g"], p["ln_e"]["b"], blk_y)
    return x_new, y_new


def kernel(edge_index, lg_edge_index, atom_feats, bondlength, cos_angles, timesteps, params):
    src, dst = edge_index[0], edge_index[1]
    lsrc, ldst = lg_edge_index[0], lg_edge_index[1]
    n = atom_feats.shape[0]
    e = bondlength.shape[0]

    egc_ps = ([lp["node"] for lp in params["alignn"]]
              + [lp["edge"] for lp in params["alignn"]]
              + list(params["gcn"])
              + [params["edges_l1"], params["edges_l2"], params["atoms_l"]])
    wp_all = jnp.concatenate([q["time_proj"]["w"] for q in egc_ps], axis=1)
    bp_all = jnp.concatenate([q["time_proj"]["b"] for q in egc_ps])
    tp_all = _time_tp(timesteps, params, len(egc_ps), wp_all, bp_all)

    x = _atom_emb(atom_feats, params["atom_emb"])
    y = _emb2(bondlength, params["edge_emb"][0], params["edge_emb"][1], 0.0, 8.0, 80, BLK_E)
    z = _emb2(cos_angles, params["angle_emb"][0], params["angle_emb"][1], -1.0, 1.0, 40, BLK_E)

    prep_n = _graph_prep(src, dst, GP_NODE)
    prep_l = _graph_prep(lsrc, ldst, GP_EDGE)

    na = len(params["alignn"])
    for i, lp in enumerate(params["alignn"]):
        x, m = _egc_layer(lp["node"], tp_all[i], prep_n, GP_NODE, x, y, BLK_N, BLK_E)
        y, z = _egc_layer(lp["edge"], tp_all[na + i], prep_l, GP_EDGE, m, z, BLK_E, BLK_E)
    for j, lp in enumerate(params["gcn"]):
        x, y = _egc_layer(lp, tp_all[2 * na + j], prep_n, GP_NODE, x, y, BLK_N, BLK_E)
    xe, ye = _egc_layer(params["edges_l1"], tp_all[9], prep_n, GP_NODE, x, y, BLK_N, BLK_E)
    xe, ye = _egc_layer(params["edges_l2"], tp_all[10], prep_n, GP_NODE, xe, ye, BLK_N, BLK_E)
    edge_out = _readout(params["edges_ro"], ye, BLK_E)
    xa, ya = _egc_layer(params["atoms_l"], tp_all[11], prep_n, GP_NODE, x, y, BLK_N, BLK_E)
    atom_out = _readout(params["atoms_ro"], xa, BLK_N)
    return jnp.concatenate([atom_out, edge_out], axis=0)


# speedup vs baseline: 1.4750x; 1.4750x over previous
import functools
import math

import jax
import jax.numpy as jnp
from jax import lax
from jax.experimental import pallas as pl
from jax.experimental.pallas import tpu as pltpu, tpu_sc as plsc

HID = 96
EMB = 64

N_NODES = 50000
N_EDGES = 800000
BLK_N = 2000
BLK_E = 3200


def _ln_silu(h, g, b):
    m = h.mean(axis=-1, keepdims=True)
    v = ((h - m) ** 2).mean(axis=-1, keepdims=True)
    h = (h - m) / jnp.sqrt(v + 1e-5) * g + b
    return h * jax.nn.sigmoid(h)


def _emb2_kernel(xs_ref, w1_ref, b1_ref, g1_ref, n1_ref, w2_ref, b2_ref,
                 g2_ref, n2_ref, o_ref, *, vmin, vmax, bins):
    xs = xs_ref[...]
    delta = (vmax - vmin) / (bins - 1)
    centers = vmin + delta * lax.broadcasted_iota(jnp.int32, (1, bins), 1).astype(jnp.float32)
    gamma = 1.0 / (delta * delta)
    r = jnp.exp(-gamma * (xs - centers) ** 2)
    h = _ln_silu(r @ w1_ref[...] + b1_ref[...], g1_ref[...], n1_ref[...])
    h = _ln_silu(h @ w2_ref[...] + b2_ref[...], g2_ref[...], n2_ref[...])
    o_ref[...] = h


def _emb2(xs, p1, p2, vmin, vmax, bins, blk):
    rows = xs.shape[0]
    d1 = p1["lin"]["w"].shape[1]
    d2 = p2["lin"]["w"].shape[1]
    f = pl.pallas_call(
        functools.partial(_emb2_kernel, vmin=vmin, vmax=vmax, bins=bins),
        grid=(rows // blk,),
        in_specs=[
            pl.BlockSpec((blk, 1), lambda i: (i, 0)),
            pl.BlockSpec((bins, d1), lambda i: (0, 0)),
            pl.BlockSpec((1, d1), lambda i: (0, 0)),
            pl.BlockSpec((1, d1), lambda i: (0, 0)),
            pl.BlockSpec((1, d1), lambda i: (0, 0)),
            pl.BlockSpec((d1, d2), lambda i: (0, 0)),
            pl.BlockSpec((1, d2), lambda i: (0, 0)),
            pl.BlockSpec((1, d2), lambda i: (0, 0)),
            pl.BlockSpec((1, d2), lambda i: (0, 0)),
        ],
        out_specs=pl.BlockSpec((blk, d2), lambda i: (i, 0)),
        out_shape=jax.ShapeDtypeStruct((rows, d2), jnp.float32),
    )
    r2 = lambda a: a.reshape(1, -1)
    return f(xs[:, None], p1["lin"]["w"], r2(p1["lin"]["b"]), r2(p1["ln"]["g"]),
             r2(p1["ln"]["b"]), p2["lin"]["w"], r2(p2["lin"]["b"]),
             r2(p2["ln"]["g"]), r2(p2["ln"]["b"]))


def _atom_emb_kernel(x_ref, w_ref, b_ref, g_ref, n_ref, o_ref):
    h = x_ref[...] @ w_ref[...] + b_ref[...]
    o_ref[...] = _ln_silu(h, g_ref[...], n_ref[...])


def _atom_emb(x, p):
    rows, din = x.shape
    f = pl.pallas_call(
        _atom_emb_kernel,
        grid=(rows // BLK_N,),
        in_specs=[
            pl.BlockSpec((BLK_N, din), lambda i: (i, 0)),
            pl.BlockSpec((din, HID), lambda i: (0, 0)),
            pl.BlockSpec((1, HID), lambda i: (0, 0)),
            pl.BlockSpec((1, HID), lambda i: (0, 0)),
            pl.BlockSpec((1, HID), lambda i: (0, 0)),
        ],
        out_specs=pl.BlockSpec((BLK_N, HID), lambda i: (i, 0)),
        out_shape=jax.ShapeDtypeStruct((rows, HID), jnp.float32),
    )
    r2 = lambda a: a.reshape(1, -1)
    return f(x, p["lin"]["w"], r2(p["lin"]["b"]), r2(p["ln"]["g"]), r2(p["ln"]["b"]))


def _time_kernel(ts_ref, w1_ref, b1_ref, g1_ref, n1_ref, w2_ref, b2_ref,
                 g2_ref, n2_ref, wp_ref, bp_ref, o_ref):
    ts = ts_ref[...]
    half = EMB // 2
    fr = math.log(10000.0) / (half - 1)
    freqs = jnp.exp(lax.broadcasted_iota(jnp.int32, (1, half), 1).astype(jnp.float32) * -fr)
    a = ts * freqs
    t = jnp.concatenate([jnp.sin(a), jnp.cos(a)], axis=1)
    t = _ln_silu(t @ w1_ref[...] + b1_ref[...], g1_ref[...], n1_ref[...])
    t = _ln_silu(t @ w2_ref[...] + b2_ref[...], g2_ref[...], n2_ref[...])
    o_ref[...] = t @ wp_ref[...] + bp_ref[...]


def _time_tp(timesteps, params, n_layers_tp, wp_all, bp_all):
    p1, p2 = params["time_emb"]
    ts8 = jnp.zeros((8, 1), jnp.float32).at[0, 0].set(timesteps[0])
    r2 = lambda a: a.reshape(1, -1)
    f = pl.pallas_call(
        _time_kernel,
        out_shape=jax.ShapeDtypeStruct((8, n_layers_tp * HID), jnp.float32),
    )
    out = f(ts8, p1["lin"]["w"], r2(p1["lin"]["b"]), r2(p1["ln"]["g"]), r2(p1["ln"]["b"]),
            p2["lin"]["w"], r2(p2["lin"]["b"]), r2(p2["ln"]["g"]), r2(p2["ln"]["b"]),
            wp_all, r2(bp_all))
    return out[0].reshape(n_layers_tp, HID)


def _pre_kernel(x_ref, w_ref, b_ref, esrc_ref, edst_ref, bh_ref, xu_ref):
    r = x_ref[...] @ w_ref[...] + b_ref[...]
    blk = r.shape[0]
    z = jnp.zeros((blk, 128 - HID), jnp.float32)
    esrc_ref[...] = jnp.concatenate([r[:, 0:96], z], axis=1)
    edst_ref[...] = jnp.concatenate([r[:, 96:192], z], axis=1)
    bh_ref[...] = jnp.concatenate([r[:, 192:288], z], axis=1)
    xu_ref[...] = r[:, 288:384]


def _egc_pre(x, wcat, bcat, blk):
    rows = x.shape[0]
    f = pl.pallas_call(
        _pre_kernel,
        grid=(rows // blk,),
        in_specs=[
            pl.BlockSpec((blk, HID), lambda i: (i, 0)),
            pl.BlockSpec((HID, 384), lambda i: (0, 0)),
            pl.BlockSpec((1, 384), lambda i: (0, 0)),
        ],
        out_specs=[
            pl.BlockSpec((blk, 128), lambda i: (i, 0)),
            pl.BlockSpec((blk, 128), lambda i: (i, 0)),
            pl.BlockSpec((blk, 128), lambda i: (i, 0)),
            pl.BlockSpec((blk, HID), lambda i: (i, 0)),
        ],
        out_shape=[
            jax.ShapeDtypeStruct((rows, 128), jnp.float32),
            jax.ShapeDtypeStruct((rows, 128), jnp.float32),
            jax.ShapeDtypeStruct((rows, 128), jnp.float32),
            jax.ShapeDtypeStruct((rows, HID), jnp.float32),
        ],
    )
    return f(x, wcat, bcat.reshape(1, -1))


def _ge_kernel(y_ref, w_ref, b_ref, o_ref):
    r = y_ref[...] @ w_ref[...] + b_ref[...]
    blk = r.shape[0]
    z = jnp.zeros((blk, 128 - HID), jnp.float32)
    o_ref[...] = jnp.concatenate([r, z], axis=1)


def _egc_ge(y, w, b, blk, out_rows):
    rows = y.shape[0]
    f = pl.pallas_call(
        _ge_kernel,
        grid=(rows // blk,),
        in_specs=[
            pl.BlockSpec((blk, HID), lambda i: (i, 0)),
            pl.BlockSpec((HID, HID), lambda i: (0, 0)),
            pl.BlockSpec((1, HID), lambda i: (0, 0)),
        ],
        out_specs=pl.BlockSpec((blk, 128), lambda i: (i, 0)),
        out_shape=jax.ShapeDtypeStruct((out_rows, 128), jnp.float32),
    )
    return f(y, w, b.reshape(1, -1))


def _post_x_kernel(x_ref, xu_ref, acc_ref, g_ref, b_ref, o_ref):
    acc = acc_ref[...]
    h = acc[:, 0:96] / (acc[:, 96:192] + 1e-6)
    xo = _ln_silu(xu_ref[...] + h, g_ref[...], b_ref[...])
    o_ref[...] = x_ref[...] + xo


def _egc_post_x(x, xu, acc, g, b, blk):
    rows = x.shape[0]
    f = pl.pallas_call(
        _post_x_kernel,
        grid=(rows // blk,),
        in_specs=[
            pl.BlockSpec((blk, HID), lambda i: (i, 0)),
            pl.BlockSpec((blk, HID), lambda i: (i, 0)),
            pl.BlockSpec((blk, 192), lambda i: (i, 0)),
            pl.BlockSpec((1, HID), lambda i: (0, 0)),
            pl.BlockSpec((1, HID), lambda i: (0, 0)),
        ],
        out_specs=pl.BlockSpec((blk, HID), lambda i: (i, 0)),
        out_shape=jax.ShapeDtypeStruct((rows, HID), jnp.float32),
    )
    return f(x, xu, acc, g.reshape(1, -1), b.reshape(1, -1))


def _post_y_kernel(y_ref, m_ref, g_ref, b_ref, o_ref):
    yo = _ln_silu(m_ref[...][:, 0:96], g_ref[...], b_ref[...])
    o_ref[...] = y_ref[...] + yo


def _egc_post_y(y, m_arr, g, b, blk):
    rows = y.shape[0]
    f = pl.pallas_call(
        _post_y_kernel,
        grid=(rows // blk,),
        in_specs=[
            pl.BlockSpec((blk, HID), lambda i: (i, 0)),
            pl.BlockSpec((blk, 128), lambda i: (i, 0)),
            pl.BlockSpec((1, HID), lambda i: (0, 0)),
            pl.BlockSpec((1, HID), lambda i: (0, 0)),
        ],
        out_specs=pl.BlockSpec((blk, HID), lambda i: (i, 0)),
        out_shape=jax.ShapeDtypeStruct((rows, HID), jnp.float32),
    )
    return f(y, m_arr, g.reshape(1, -1), b.reshape(1, -1))


def _readout_kernel(x_ref, w_ref, b_ref, o_ref):
    o_ref[...] = x_ref[...] @ w_ref[...] + b_ref[...]


def _readout(p, x, blk):
    rows = x.shape[0]
    f = pl.pallas_call(
        _readout_kernel,
        grid=(rows // blk,),
        in_specs=[
            pl.BlockSpec((blk, HID), lambda i: (i, 0)),
            pl.BlockSpec((HID, 1), lambda i: (0, 0)),
            pl.BlockSpec((1, 1), lambda i: (0, 0)),
        ],
        out_specs=pl.BlockSpec((blk, 1), lambda i: (i, 0)),
        out_shape=jax.ShapeDtypeStruct((rows, 1), jnp.float32),
    )
    return f(x, p["w"], p["b"].reshape(1, 1))


NC, NS, L = 2, 16, 16
NW = NC * NS
EG = N_EDGES
PCH = 2048
NCHUNKS = (EG + PCH - 1) // PCH
LAST_N = EG - (NCHUNKS - 1) * PCH
KC = 128

GP_NODE = (128, 7, 416, 13, 416 * 128)
GP_EDGE = (256, 8, 3136, 98, 3136 * 256)
NB_HALF_MAX = 1568
TRASH = EG + PCH

_MESH = plsc.VectorSubcoreMesh(core_axis_name="c", subcore_axis_name="s")


def _wid():
    return lax.axis_index("s") * NC + lax.axis_index("c")


def _lane_iota():
    return lax.iota(jnp.int32, L)


def _sel_lane(vec, k):
    sc = vec[0]
    for l in range(1, L):
        sc = jnp.where(k == l, vec[l], sc)
    return sc


def _hist_body(shift, nb, hb, dst_hbm, cnt_hbm, dst_v, cnt_v, hist_s):
    w = _wid()

    def _z(i, _):
        hist_s[i] = 0
        return 0
    lax.fori_loop(0, nb, _z, 0)

    nrounds = (NCHUNKS - w + NW - 1) // NW

    def _round(k, _):
        c = w + k * NW
        cs = c * PCH

        @pl.when(c < NCHUNKS - 1)
        def _():
            pltpu.sync_copy(dst_hbm.at[pl.ds(cs, PCH)], dst_v)

        @pl.when(c == NCHUNKS - 1)
        def _():
            pltpu.sync_copy(dst_hbm.at[pl.ds(cs, LAST_N)], dst_v.at[pl.ds(0, LAST_N)])

        ng = jnp.where(c == NCHUNKS - 1, LAST_N // L, PCH // L)

        def _grp(g, _):
            b16 = lax.shift_right_logical(dst_v[pl.ds(g * L, L)], shift) - hb
            for l in range(L):
                b = b16[l]

                @pl.when((b >= 0) & (b < nb))
                def _():
                    hist_s[b] = hist_s[b] + 1
            return 0
        lax.fori_loop(0, ng, _grp, 0)
        return 0
    lax.fori_loop(0, nrounds, _round, 0)

    def _flush(g, _):
        v = jnp.zeros((L,), jnp.int32)
        io = _lane_iota()
        for l in range(L):
            v = jnp.where(io == l, hist_s[g * L + l], v)
        cnt_v[pl.ds(g * L, L)] = v
        return 0
    lax.fori_loop(0, nb // L, _flush, 0)
    pltpu.sync_copy(cnt_v, cnt_hbm.at[w])


def _prep_hist(dst, gp, hb, nb):
    _, shift, _, _, _ = gp
    f = pl.kernel(
        functools.partial(_hist_body, shift, nb, hb),
        out_type=[jax.ShapeDtypeStruct((NW, nb), jnp.int32)],
        mesh=_MESH,
        name="prep_hist",
        scratch_types=[
            pltpu.VMEM((PCH,), jnp.int32),
            pltpu.VMEM((nb,), jnp.int32),
            pltpu.SMEM((nb,), jnp.int32),
        ],
    )
    return f(dst)[0]


def _scat_body(shift, nb, hb, dst_hbm, src_hbm, cnt_hbm, base_hbm,
               perm_hbm, srcp_hbm, dstp_hbm, bst_hbm,
               cnt_v, dst_v, src_v, bst_v, base_v,
               pos_b, id_b, src_b, dst_b, off_s, sem):
    w = _wid()
    io = _lane_iota()
    pltpu.sync_copy(cnt_hbm, cnt_v)
    pltpu.sync_copy(base_hbm, base_v)
    base0 = base_v[pl.ds(0, L)][0]

    def _off(b, base):
        cv0 = cnt_v[pl.ds(b * NW, L)]
        cv1 = cnt_v[pl.ds(b * NW + L, L)]
        excl = jnp.int32(0)
        tot = jnp.int32(0)
        for l in range(L):
            el = cv0[l]
            excl = excl + jnp.where(w > l, el, 0)
            tot = tot + el
        for l in range(L):
            el = cv1[l]
            excl = excl + jnp.where(w > L + l, el, 0)
            tot = tot + el
        off_s[b] = base + excl
        return base + tot
    lax.fori_loop(0, nb, _off, base0)

    @pl.when(w == 0)
    def _():
        def _fl(g, _):
            v = jnp.zeros((L,), jnp.int32)
            for l in range(L):
                v = jnp.where(io == l, off_s[g * L + l], v)
            bst_v[pl.ds(g * L, L)] = v
            return 0
        lax.fori_loop(0, nb // L, _fl, 0)
        pltpu.sync_copy(bst_v, bst_hbm)

    nrounds = (NCHUNKS - w + NW - 1) // NW

    def _round(k, _):
        c = w + k * NW
        cs = c * PCH

        @pl.when(c < NCHUNKS - 1)
        def _():
            pltpu.sync_copy(dst_hbm.at[pl.ds(cs, PCH)], dst_v)
            pltpu.sync_copy(src_hbm.at[pl.ds(cs, PCH)], src_v)

        @pl.when(c == NCHUNKS - 1)
        def _():
            pltpu.sync_copy(dst_hbm.at[pl.ds(cs, LAST_N)], dst_v.at[pl.ds(0, LAST_N)])
            pltpu.sync_copy(src_hbm.at[pl.ds(cs, LAST_N)], src_v.at[pl.ds(0, LAST_N)])

        ngg = jnp.where(c == NCHUNKS - 1, LAST_N // (8 * L), PCH // (8 * L))

        def _row(gg, _):
            for q in range(8):
                o = gg * 8 * L + q * L
                d16 = dst_v[pl.ds(o, L)]
                s16 = src_v[pl.ds(o, L)]
                b16 = lax.shift_right_logical(d16, shift) - hb
                id16 = io + (cs + o)
                posv = TRASH + o + io
                for l in range(L):
                    b = b16[l]
                    inh = (b >= 0) & (b < nb)
                    bc = jnp.clip(b, 0, nb - 1)

                    @pl.when(inh)
                    def _():
                        off_s[bc] = off_s[bc] + 1

                    p2 = jnp.where(inh, off_s[bc] - 1, TRASH + o + l)
                    posv = jnp.where(io == l, p2, posv)
                pos_b[gg, pl.ds(q * L, L)] = posv
                id_b[gg, pl.ds(q * L, L)] = id16
                src_b[gg, pl.ds(q * L, L)] = s16
                dst_b[gg, pl.ds(q * L, L)] = d16
            return 0
        lax.fori_loop(0, ngg, _row, 0)

        def _scat_row(j, _):
            pltpu.async_copy(id_b.at[j], perm_hbm.at[pos_b.at[j]], sem).wait()
            pltpu.async_copy(src_b.at[j], srcp_hbm.at[pos_b.at[j]], sem).wait()
            pltpu.async_copy(dst_b.at[j], dstp_hbm.at[pos_b.at[j]], sem).wait()
            return 0
        lax.fori_loop(0, ngg, _scat_row, 0)
        return 0
    lax.fori_loop(0, nrounds, _round, 0)

    @pl.when(w == NW - 1)
    def _():
        def _pv(g, _):
            dst_v[pl.ds(g * L, L)] = jnp.full((L,), EG, jnp.int32)
            src_v[pl.ds(g * L, L)] = jnp.zeros((L,), jnp.int32)
            return 0
        lax.fori_loop(0, PCH // L, _pv, 0)
        pltpu.sync_copy(dst_v, perm_hbm.at[pl.ds(EG, PCH)])
        pltpu.sync_copy(src_v, srcp_hbm.at[pl.ds(EG, PCH)])
        pltpu.sync_copy(src_v, dstp_hbm.at[pl.ds(EG, PCH)])


def _prep_scatter(dst, src, cnt_t, base0, gp, hb, nb):
    _, shift, _, _, _ = gp
    f = pl.kernel(
        functools.partial(_scat_body, shift, nb, hb),
        out_type=[jax.ShapeDtypeStruct((EG + 2 * PCH + 2048,), jnp.int32),
                  jax.ShapeDtypeStruct((EG + 2 * PCH + 2048,), jnp.int32),
                  jax.ShapeDtypeStruct((EG + 2 * PCH + 2048,), jnp.int32),
                  jax.ShapeDtypeStruct((nb,), jnp.int32)],
        mesh=_MESH,
        name="prep_scat",
        scratch_types=[
            pltpu.VMEM((nb * NW,), jnp.int32),
            pltpu.VMEM((PCH,), jnp.int32),
            pltpu.VMEM((PCH,), jnp.int32),
            pltpu.VMEM((nb,), jnp.int32),
            pltpu.VMEM((L,), jnp.int32),
            pltpu.VMEM((PCH // (8 * L), 8 * L), jnp.int32),
            pltpu.VMEM((PCH // (8 * L), 8 * L), jnp.int32),
            pltpu.VMEM((PCH // (8 * L), 8 * L), jnp.int32),
            pltpu.VMEM((PCH // (8 * L), 8 * L), jnp.int32),
            pltpu.SMEM((nb,), jnp.int32),
            pltpu.SemaphoreType.DMA,
        ],
    )
    return f(dst, src, cnt_t.reshape(-1), base0)


def _graph_prep(src, dst, gp):
    _, _, b_pad, _, _ = gp
    zero16 = jnp.zeros((L,), jnp.int32)
    if b_pad <= NB_HALF_MAX:
        cnt = _prep_hist(dst, gp, 0, b_pad)
        perm, srcp, dstp, bst = _prep_scatter(
            dst, src, jnp.transpose(cnt), zero16, gp, 0, b_pad)
        bstart = jnp.concatenate([bst, jnp.full((L,), EG, jnp.int32)])
        return perm, srcp, dstp, bstart
    nb = b_pad // 2
    cnt1 = _prep_hist(dst, gp, 0, nb)
    cnt2 = _prep_hist(dst, gp, nb, nb)
    n1 = jnp.sum(cnt1).astype(jnp.int32)
    base2 = zero16.at[0].set(n1)
    p1, s1, d1, bst1 = _prep_scatter(dst, src, jnp.transpose(cnt1), zero16, gp, 0, nb)
    p2, s2, d2, bst2 = _prep_scatter(dst, src, jnp.transpose(cnt2), base2, gp, nb, nb)
    pos = lax.broadcasted_iota(jnp.int32, p1.shape, 0)
    first = pos < n1
    perm = jnp.where(first, p1, p2)
    srcp = jnp.where(first, s1, s2)
    dstp = jnp.where(first, d1, d2)
    bstart = jnp.concatenate([bst1, bst2, jnp.full((L,), EG, jnp.int32)])
    return perm, srcp, dstp, bstart


def _egc_sc_body(w_bucket, b_pad, bpw, ge_rows,
                 esrc_hbm, edst_hbm, bh_hbm, ge_hbm,
                 perm_hbm, srcp_hbm, dstp_hbm, bst_hbm,
                 m_hbm, acc_hbm,
                 pc, sc_, dc, m_buf, bh_buf, acc, bst16, sema, semb, semc):
    w = _wid()
    io = _lane_iota()
    zf = jnp.zeros((L,), jnp.float32)

    def _bucket(bi, _):
        b = w * bpw + bi
        al = (b // 8) * 8
        pltpu.sync_copy(bst_hbm.at[pl.ds(al, L)], bst16)
        bv = bst16[pl.ds(0, L)]
        e0 = _sel_lane(bv, b - al)
        e1 = _sel_lane(bv, b - al + 1)

        def _zr(r, _):
            for j in range(12):
                acc[r, pl.ds(j * L, L)] = zf
            return 0
        lax.fori_loop(0, w_bucket, _zr, 0)

        cs0 = (e0 // 8) * 8
        nch = (e1 - cs0 + KC - 1) // KC

        def _chunk(ci, _):
            cs = cs0 + ci * KC
            lo = jnp.maximum(e0 - cs, 0)
            hi = jnp.minimum(e1 - cs, KC)
            i1 = pltpu.async_copy(perm_hbm.at[pl.ds(cs, KC)], pc, semb)
            i2 = pltpu.async_copy(srcp_hbm.at[pl.ds(cs, KC)], sc_, semb)
            i3 = pltpu.async_copy(dstp_hbm.at[pl.ds(cs, KC)], dc, semb)
            i1.wait()
            i2.wait()
            i3.wait()
            g1 = pltpu.async_copy(ge_hbm.at[pc], m_buf, sema)
            g2 = pltpu.async_copy(bh_hbm.at[sc_], bh_buf, semb)
            g1.wait()
            d1 = pltpu.async_copy(esrc_hbm.at[sc_], m_buf, sema, add=True)
            d2 = pltpu.async_copy(edst_hbm.at[dc], m_buf, sema, add=True)
            d1.wait()
            d2.wait()
            g2.wait()
            pltpu.async_copy(m_buf, m_hbm.at[pc], semc).wait()

            @plsc.parallel_loop(0, KC, unroll=4)
            def _sig(i):
                for j in range(6):
                    m = m_buf[i, pl.ds(j * L, L)]
                    sg = 1.0 / (1.0 + jnp.exp(-m))
                    bhv = bh_buf[i, pl.ds(j * L, L)]
                    m_buf[i, pl.ds(j * L, L)] = sg
                    bh_buf[i, pl.ds(j * L, L)] = sg * bhv

            bw = b * w_bucket
            for q in range(KC // L):
                dls = dc[pl.ds(q * L, L)] - bw
                for l in range(L):
                    i = q * L + l
                    dl = dls[l]

                    @pl.when((i >= lo) & (i < hi))
                    def _():
                        for j in range(6):
                            plsc.addupdate(acc.at[dl, pl.ds(j * L, L)],
                                           bh_buf[i, pl.ds(j * L, L)])
                            plsc.addupdate(acc.at[dl, pl.ds(96 + j * L, L)],
                                           m_buf[i, pl.ds(j * L, L)])
            return 0
        lax.fori_loop(0, nch, _chunk, 0)
        pltpu.sync_copy(acc, acc_hbm.at[pl.ds(b * w_bucket, w_bucket)])
        return 0
    lax.fori_loop(0, bpw, _bucket, 0)


def _egc_edge_sc(esrc, edst, bh, ge, prep, gp):
    perm, srcp, dstp, bst = prep
    w_bucket, _, b_pad, bpw, s_pad = gp
    f = pl.kernel(
        functools.partial(_egc_sc_body, w_bucket, b_pad, bpw, ge.shape[0]),
        out_type=[jax.ShapeDtypeStruct((EG + PCH, 128), jnp.float32),
                  jax.ShapeDtypeStruct((s_pad, 192), jnp.float32)],
        mesh=_MESH,
        name="egc_edge",
        scratch_types=[
            pltpu.VMEM((KC,), jnp.int32),
            pltpu.VMEM((KC,), jnp.int32),
            pltpu.VMEM((KC,), jnp.int32),
            pltpu.VMEM((KC, 128), jnp.float32),
            pltpu.VMEM((KC, 128), jnp.float32),
            pltpu.VMEM((w_bucket, 192), jnp.float32),
            pltpu.VMEM((L,), jnp.int32),
            pltpu.SemaphoreType.DMA,
            pltpu.SemaphoreType.DMA,
            pltpu.SemaphoreType.DMA,
        ],
    )
    return f(esrc, edst, bh, ge, perm, srcp, dstp, bst)


def _edge_phase(esrc, edst, bh, ge, prep, gp):
    m_arr, acc = _egc_edge_sc(esrc, edst, bh, ge, prep, gp)
    return m_arr, acc


def _egc_layer(p, tp, prep, gp, x, y, blk_x, blk_y):
    wcat = jnp.concatenate([p["src_gate"]["w"], p["dst_gate"]["w"],
                            p["dst_update"]["w"], p["src_update"]["w"]], axis=1)
    bcat = jnp.concatenate([p["src_gate"]["b"] + tp, p["dst_gate"]["b"],
                            p["dst_update"]["b"], p["src_update"]["b"]])
    esrc, edst, bh, xu = _egc_pre(x, wcat, bcat, blk_x)
    ge = _egc_ge(y, p["edge_gate"]["w"], p["edge_gate"]["b"], blk_y, EG + 8)
    m_arr, acc = _edge_phase(esrc, edst, bh, ge, prep, gp)
    x_new = _egc_post_x(x, xu, acc, p["ln_n"]["g"], p["ln_n"]["b"], blk_x)
    y_new = _egc_post_y(y, m_arr, p["ln_e"]["g"], p["ln_e"]["b"], blk_y)
    return x_new, y_new


def kernel(edge_index, lg_edge_index, atom_feats, bondlength, cos_angles, timesteps, params):
    src, dst = edge_index[0], edge_index[1]
    lsrc, ldst = lg_edge_index[0], lg_edge_index[1]
    n = atom_feats.shape[0]
    e = bondlength.shape[0]

    egc_ps = ([lp["node"] for lp in params["alignn"]]
              + [lp["edge"] for lp in params["alignn"]]
              + list(params["gcn"])
              + [params["edges_l1"], params["edges_l2"], params["atoms_l"]])
    wp_all = jnp.concatenate([q["time_proj"]["w"] for q in egc_ps], axis=1)
    bp_all = jnp.concatenate([q["time_proj"]["b"] for q in egc_ps])
    tp_all = _time_tp(timesteps, params, len(egc_ps), wp_all, bp_all)

    x = _atom_emb(atom_feats, params["atom_emb"])
    y = _emb2(bondlength, params["edge_emb"][0], params["edge_emb"][1], 0.0, 8.0, 80, BLK_E)
    z = _emb2(cos_angles, params["angle_emb"][0], params["angle_emb"][1], -1.0, 1.0, 40, BLK_E)

    prep_n = _graph_prep(src, dst, GP_NODE)
    prep_l = _graph_prep(lsrc, ldst, GP_EDGE)

    na = len(params["alignn"])
    for i, lp in enumerate(params["alignn"]):
        x, m = _egc_layer(lp["node"], tp_all[i], prep_n, GP_NODE, x, y, BLK_N, BLK_E)
        y, z = _egc_layer(lp["edge"], tp_all[na + i], prep_l, GP_EDGE, m, z, BLK_E, BLK_E)
    for j, lp in enumerate(params["gcn"]):
        x, y = _egc_layer(lp, tp_all[2 * na + j], prep_n, GP_NODE, x, y, BLK_N, BLK_E)
    xe, ye = _egc_layer(params["edges_l1"], tp_all[9], prep_n, GP_NODE, x, y, BLK_N, BLK_E)
    xe, ye = _egc_layer(params["edges_l2"], tp_all[10], prep_n, GP_NODE, xe, ye, BLK_N, BLK_E)
    edge_out = _readout(params["edges_ro"], ye, BLK_E)
    xa, ya = _egc_layer(params["atoms_l"], tp_all[11], prep_n, GP_NODE, x, y, BLK_N, BLK_E)
    atom_out = _readout(params["atoms_ro"], xa, BLK_N)
    return jnp.concatenate([atom_out, edge_out], axis=0)


# speedup vs baseline: 1.5339x; 1.0399x over previous
import functools
import math

import jax
import jax.numpy as jnp
from jax import lax
from jax.experimental import pallas as pl
from jax.experimental.pallas import tpu as pltpu, tpu_sc as plsc

HID = 96
EMB = 64

N_NODES = 50000
N_EDGES = 800000
BLK_N = 2000
BLK_E = 3200


def _ln_silu(h, g, b):
    m = h.mean(axis=-1, keepdims=True)
    v = ((h - m) ** 2).mean(axis=-1, keepdims=True)
    h = (h - m) / jnp.sqrt(v + 1e-5) * g + b
    return h * jax.nn.sigmoid(h)


def _emb2_kernel(xs_ref, w1_ref, b1_ref, g1_ref, n1_ref, w2_ref, b2_ref,
                 g2_ref, n2_ref, o_ref, *, vmin, vmax, bins):
    xs = xs_ref[...]
    delta = (vmax - vmin) / (bins - 1)
    centers = vmin + delta * lax.broadcasted_iota(jnp.int32, (1, bins), 1).astype(jnp.float32)
    gamma = 1.0 / (delta * delta)
    r = jnp.exp(-gamma * (xs - centers) ** 2)
    h = _ln_silu(r @ w1_ref[...] + b1_ref[...], g1_ref[...], n1_ref[...])
    h = _ln_silu(h @ w2_ref[...] + b2_ref[...], g2_ref[...], n2_ref[...])
    o_ref[...] = h


def _emb2(xs, p1, p2, vmin, vmax, bins, blk):
    rows = xs.shape[0]
    d1 = p1["lin"]["w"].shape[1]
    d2 = p2["lin"]["w"].shape[1]
    f = pl.pallas_call(
        functools.partial(_emb2_kernel, vmin=vmin, vmax=vmax, bins=bins),
        grid=(rows // blk,),
        in_specs=[
            pl.BlockSpec((blk, 1), lambda i: (i, 0)),
            pl.BlockSpec((bins, d1), lambda i: (0, 0)),
            pl.BlockSpec((1, d1), lambda i: (0, 0)),
            pl.BlockSpec((1, d1), lambda i: (0, 0)),
            pl.BlockSpec((1, d1), lambda i: (0, 0)),
            pl.BlockSpec((d1, d2), lambda i: (0, 0)),
            pl.BlockSpec((1, d2), lambda i: (0, 0)),
            pl.BlockSpec((1, d2), lambda i: (0, 0)),
            pl.BlockSpec((1, d2), lambda i: (0, 0)),
        ],
        out_specs=pl.BlockSpec((blk, d2), lambda i: (i, 0)),
        out_shape=jax.ShapeDtypeStruct((rows, d2), jnp.float32),
    )
    r2 = lambda a: a.reshape(1, -1)
    return f(xs[:, None], p1["lin"]["w"], r2(p1["lin"]["b"]), r2(p1["ln"]["g"]),
             r2(p1["ln"]["b"]), p2["lin"]["w"], r2(p2["lin"]["b"]),
             r2(p2["ln"]["g"]), r2(p2["ln"]["b"]))


def _atom_emb_kernel(x_ref, w_ref, b_ref, g_ref, n_ref, o_ref):
    h = x_ref[...] @ w_ref[...] + b_ref[...]
    o_ref[...] = _ln_silu(h, g_ref[...], n_ref[...])


def _atom_emb(x, p):
    rows, din = x.shape
    f = pl.pallas_call(
        _atom_emb_kernel,
        grid=(rows // BLK_N,),
        in_specs=[
            pl.BlockSpec((BLK_N, din), lambda i: (i, 0)),
            pl.BlockSpec((din, HID), lambda i: (0, 0)),
            pl.BlockSpec((1, HID), lambda i: (0, 0)),
            pl.BlockSpec((1, HID), lambda i: (0, 0)),
            pl.BlockSpec((1, HID), lambda i: (0, 0)),
        ],
        out_specs=pl.BlockSpec((BLK_N, HID), lambda i: (i, 0)),
        out_shape=jax.ShapeDtypeStruct((rows, HID), jnp.float32),
    )
    r2 = lambda a: a.reshape(1, -1)
    return f(x, p["lin"]["w"], r2(p["lin"]["b"]), r2(p["ln"]["g"]), r2(p["ln"]["b"]))


def _time_kernel(ts_ref, w1_ref, b1_ref, g1_ref, n1_ref, w2_ref, b2_ref,
                 g2_ref, n2_ref, wp_ref, bp_ref, o_ref):
    ts = ts_ref[...]
    half = EMB // 2
    fr = math.log(10000.0) / (half - 1)
    freqs = jnp.exp(lax.broadcasted_iota(jnp.int32, (1, half), 1).astype(jnp.float32) * -fr)
    a = ts * freqs
    t = jnp.concatenate([jnp.sin(a), jnp.cos(a)], axis=1)
    t = _ln_silu(t @ w1_ref[...] + b1_ref[...], g1_ref[...], n1_ref[...])
    t = _ln_silu(t @ w2_ref[...] + b2_ref[...], g2_ref[...], n2_ref[...])
    o_ref[...] = t @ wp_ref[...] + bp_ref[...]


def _time_tp(timesteps, params, n_layers_tp, wp_all, bp_all):
    p1, p2 = params["time_emb"]
    ts8 = jnp.zeros((8, 1), jnp.float32).at[0, 0].set(timesteps[0])
    r2 = lambda a: a.reshape(1, -1)
    f = pl.pallas_call(
        _time_kernel,
        out_shape=jax.ShapeDtypeStruct((8, n_layers_tp * HID), jnp.float32),
    )
    out = f(ts8, p1["lin"]["w"], r2(p1["lin"]["b"]), r2(p1["ln"]["g"]), r2(p1["ln"]["b"]),
            p2["lin"]["w"], r2(p2["lin"]["b"]), r2(p2["ln"]["g"]), r2(p2["ln"]["b"]),
            wp_all, r2(bp_all))
    return out[0].reshape(n_layers_tp, HID)


def _pre_kernel(x_ref, w_ref, b_ref, esrc_ref, edst_ref, bh_ref, xu_ref):
    r = x_ref[...] @ w_ref[...] + b_ref[...]
    blk = r.shape[0]
    z = jnp.zeros((blk, 128 - HID), jnp.float32)
    esrc_ref[...] = jnp.concatenate([r[:, 0:96], z], axis=1)
    edst_ref[...] = jnp.concatenate([r[:, 96:192], z], axis=1)
    bh_ref[...] = jnp.concatenate([r[:, 192:288], z], axis=1)
    xu_ref[...] = r[:, 288:384]


def _egc_pre(x, wcat, bcat, blk):
    rows = x.shape[0]
    f = pl.pallas_call(
        _pre_kernel,
        grid=(rows // blk,),
        in_specs=[
            pl.BlockSpec((blk, HID), lambda i: (i, 0)),
            pl.BlockSpec((HID, 384), lambda i: (0, 0)),
            pl.BlockSpec((1, 384), lambda i: (0, 0)),
        ],
        out_specs=[
            pl.BlockSpec((blk, 128), lambda i: (i, 0)),
            pl.BlockSpec((blk, 128), lambda i: (i, 0)),
            pl.BlockSpec((blk, 128), lambda i: (i, 0)),
            pl.BlockSpec((blk, HID), lambda i: (i, 0)),
        ],
        out_shape=[
            jax.ShapeDtypeStruct((rows, 128), jnp.float32),
            jax.ShapeDtypeStruct((rows, 128), jnp.float32),
            jax.ShapeDtypeStruct((rows, 128), jnp.float32),
            jax.ShapeDtypeStruct((rows, HID), jnp.float32),
        ],
    )
    return f(x, wcat, bcat.reshape(1, -1))


def _ge_kernel(y_ref, w_ref, b_ref, o_ref):
    r = y_ref[...] @ w_ref[...] + b_ref[...]
    blk = r.shape[0]
    z = jnp.zeros((blk, 128 - HID), jnp.float32)
    o_ref[...] = jnp.concatenate([r, z], axis=1)


def _egc_ge(y, w, b, blk, out_rows):
    rows = y.shape[0]
    f = pl.pallas_call(
        _ge_kernel,
        grid=(rows // blk,),
        in_specs=[
            pl.BlockSpec((blk, HID), lambda i: (i, 0)),
            pl.BlockSpec((HID, HID), lambda i: (0, 0)),
            pl.BlockSpec((1, HID), lambda i: (0, 0)),
        ],
        out_specs=pl.BlockSpec((blk, 128), lambda i: (i, 0)),
        out_shape=jax.ShapeDtypeStruct((out_rows, 128), jnp.float32),
    )
    return f(y, w, b.reshape(1, -1))


def _post_x_kernel(x_ref, xu_ref, acc_ref, g_ref, b_ref, o_ref):
    acc = acc_ref[...]
    h = acc[:, 0:96] / (acc[:, 96:192] + 1e-6)
    xo = _ln_silu(xu_ref[...] + h, g_ref[...], b_ref[...])
    o_ref[...] = x_ref[...] + xo


def _egc_post_x(x, xu, acc, g, b, blk):
    rows = x.shape[0]
    f = pl.pallas_call(
        _post_x_kernel,
        grid=(rows // blk,),
        in_specs=[
            pl.BlockSpec((blk, HID), lambda i: (i, 0)),
            pl.BlockSpec((blk, HID), lambda i: (i, 0)),
            pl.BlockSpec((blk, 192), lambda i: (i, 0)),
            pl.BlockSpec((1, HID), lambda i: (0, 0)),
            pl.BlockSpec((1, HID), lambda i: (0, 0)),
        ],
        out_specs=pl.BlockSpec((blk, HID), lambda i: (i, 0)),
        out_shape=jax.ShapeDtypeStruct((rows, HID), jnp.float32),
    )
    return f(x, xu, acc, g.reshape(1, -1), b.reshape(1, -1))


def _post_y_kernel(y_ref, m_ref, g_ref, b_ref, o_ref):
    yo = _ln_silu(m_ref[...][:, 0:96], g_ref[...], b_ref[...])
    o_ref[...] = y_ref[...] + yo


def _egc_post_y(y, m_arr, g, b, blk):
    rows = y.shape[0]
    f = pl.pallas_call(
        _post_y_kernel,
        grid=(rows // blk,),
        in_specs=[
            pl.BlockSpec((blk, HID), lambda i: (i, 0)),
            pl.BlockSpec((blk, 128), lambda i: (i, 0)),
            pl.BlockSpec((1, HID), lambda i: (0, 0)),
            pl.BlockSpec((1, HID), lambda i: (0, 0)),
        ],
        out_specs=pl.BlockSpec((blk, HID), lambda i: (i, 0)),
        out_shape=jax.ShapeDtypeStruct((rows, HID), jnp.float32),
    )
    return f(y, m_arr, g.reshape(1, -1), b.reshape(1, -1))


def _readout_kernel(x_ref, w_ref, b_ref, o_ref):
    o_ref[...] = x_ref[...] @ w_ref[...] + b_ref[...]


def _readout(p, x, blk):
    rows = x.shape[0]
    f = pl.pallas_call(
        _readout_kernel,
        grid=(rows // blk,),
        in_specs=[
            pl.BlockSpec((blk, HID), lambda i: (i, 0)),
            pl.BlockSpec((HID, 1), lambda i: (0, 0)),
            pl.BlockSpec((1, 1), lambda i: (0, 0)),
        ],
        out_specs=pl.BlockSpec((blk, 1), lambda i: (i, 0)),
        out_shape=jax.ShapeDtypeStruct((rows, 1), jnp.float32),
    )
    return f(x, p["w"], p["b"].reshape(1, 1))


NC, NS, L = 2, 16, 16
NW = NC * NS
EG = N_EDGES
PCH = 2048
NCHUNKS = (EG + PCH - 1) // PCH
LAST_N = EG - (NCHUNKS - 1) * PCH
KC = 96

GP_NODE = (128, 7, 416, 13, 416 * 128)
GP_EDGE = (256, 8, 3136, 98, 3136 * 256)
NB_HALF_MAX = 1568
TRASH = EG + PCH

_MESH = plsc.VectorSubcoreMesh(core_axis_name="c", subcore_axis_name="s")


def _wid():
    return lax.axis_index("s") * NC + lax.axis_index("c")


def _lane_iota():
    return lax.iota(jnp.int32, L)


def _sel_lane(vec, k):
    sc = vec[0]
    for l in range(1, L):
        sc = jnp.where(k == l, vec[l], sc)
    return sc


def _hist_body(shift, nb, hb, dst_hbm, cnt_hbm, dst_v, cnt_v, hist_s):
    w = _wid()

    def _z(i, _):
        hist_s[i] = 0
        return 0
    lax.fori_loop(0, nb, _z, 0)

    nrounds = (NCHUNKS - w + NW - 1) // NW

    def _round(k, _):
        c = w + k * NW
        cs = c * PCH

        @pl.when(c < NCHUNKS - 1)
        def _():
            pltpu.sync_copy(dst_hbm.at[pl.ds(cs, PCH)], dst_v)

        @pl.when(c == NCHUNKS - 1)
        def _():
            pltpu.sync_copy(dst_hbm.at[pl.ds(cs, LAST_N)], dst_v.at[pl.ds(0, LAST_N)])

        ng = jnp.where(c == NCHUNKS - 1, LAST_N // L, PCH // L)

        def _grp(g, _):
            b16 = lax.shift_right_logical(dst_v[pl.ds(g * L, L)], shift) - hb
            for l in range(L):
                b = b16[l]

                @pl.when((b >= 0) & (b < nb))
                def _():
                    hist_s[b] = hist_s[b] + 1
            return 0
        lax.fori_loop(0, ng, _grp, 0)
        return 0
    lax.fori_loop(0, nrounds, _round, 0)

    def _flush(g, _):
        v = jnp.zeros((L,), jnp.int32)
        io = _lane_iota()
        for l in range(L):
            v = jnp.where(io == l, hist_s[g * L + l], v)
        cnt_v[pl.ds(g * L, L)] = v
        return 0
    lax.fori_loop(0, nb // L, _flush, 0)
    pltpu.sync_copy(cnt_v, cnt_hbm.at[w])


def _prep_hist(dst, gp, hb, nb):
    _, shift, _, _, _ = gp
    f = pl.kernel(
        functools.partial(_hist_body, shift, nb, hb),
        out_type=[jax.ShapeDtypeStruct((NW, nb), jnp.int32)],
        mesh=_MESH,
        name="prep_hist",
        scratch_types=[
            pltpu.VMEM((PCH,), jnp.int32),
            pltpu.VMEM((nb,), jnp.int32),
            pltpu.SMEM((nb,), jnp.int32),
        ],
    )
    return f(dst)[0]


def _scat_body(shift, nb, hb, dst_hbm, src_hbm, cnt_hbm, base_hbm,
               perm_hbm, srcp_hbm, dstp_hbm, bst_hbm,
               cnt_v, dst_v, src_v, bst_v, base_v,
               pos_b, id_b, src_b, dst_b, off_s, sem):
    w = _wid()
    io = _lane_iota()
    pltpu.sync_copy(cnt_hbm, cnt_v)
    pltpu.sync_copy(base_hbm, base_v)
    base0 = base_v[pl.ds(0, L)][0]

    def _off(b, base):
        cv0 = cnt_v[pl.ds(b * NW, L)]
        cv1 = cnt_v[pl.ds(b * NW + L, L)]
        excl = jnp.int32(0)
        tot = jnp.int32(0)
        for l in range(L):
            el = cv0[l]
            excl = excl + jnp.where(w > l, el, 0)
            tot = tot + el
        for l in range(L):
            el = cv1[l]
            excl = excl + jnp.where(w > L + l, el, 0)
            tot = tot + el
        off_s[b] = base + excl
        return base + tot
    lax.fori_loop(0, nb, _off, base0)

    @pl.when(w == 0)
    def _():
        def _fl(g, _):
            v = jnp.zeros((L,), jnp.int32)
            for l in range(L):
                v = jnp.where(io == l, off_s[g * L + l], v)
            bst_v[pl.ds(g * L, L)] = v
            return 0
        lax.fori_loop(0, nb // L, _fl, 0)
        pltpu.sync_copy(bst_v, bst_hbm)

    nrounds = (NCHUNKS - w + NW - 1) // NW

    def _round(k, _):
        c = w + k * NW
        cs = c * PCH

        @pl.when(c < NCHUNKS - 1)
        def _():
            pltpu.sync_copy(dst_hbm.at[pl.ds(cs, PCH)], dst_v)
            pltpu.sync_copy(src_hbm.at[pl.ds(cs, PCH)], src_v)

        @pl.when(c == NCHUNKS - 1)
        def _():
            pltpu.sync_copy(dst_hbm.at[pl.ds(cs, LAST_N)], dst_v.at[pl.ds(0, LAST_N)])
            pltpu.sync_copy(src_hbm.at[pl.ds(cs, LAST_N)], src_v.at[pl.ds(0, LAST_N)])

        ngg = jnp.where(c == NCHUNKS - 1, LAST_N // (8 * L), PCH // (8 * L))

        def _row(gg, _):
            for q in range(8):
                o = gg * 8 * L + q * L
                d16 = dst_v[pl.ds(o, L)]
                s16 = src_v[pl.ds(o, L)]
                b16 = lax.shift_right_logical(d16, shift) - hb
                id16 = io + (cs + o)
                posv = TRASH + o + io
                for l in range(L):
                    b = b16[l]
                    inh = (b >= 0) & (b < nb)
                    bc = jnp.clip(b, 0, nb - 1)

                    @pl.when(inh)
                    def _():
                        off_s[bc] = off_s[bc] + 1

                    p2 = jnp.where(inh, off_s[bc] - 1, TRASH + o + l)
                    posv = jnp.where(io == l, p2, posv)
                pos_b[gg, pl.ds(q * L, L)] = posv
                id_b[gg, pl.ds(q * L, L)] = id16
                src_b[gg, pl.ds(q * L, L)] = s16
                dst_b[gg, pl.ds(q * L, L)] = d16
            return 0
        lax.fori_loop(0, ngg, _row, 0)

        def _scat_row(j, _):
            pltpu.async_copy(id_b.at[j], perm_hbm.at[pos_b.at[j]], sem).wait()
            pltpu.async_copy(src_b.at[j], srcp_hbm.at[pos_b.at[j]], sem).wait()
            pltpu.async_copy(dst_b.at[j], dstp_hbm.at[pos_b.at[j]], sem).wait()
            return 0
        lax.fori_loop(0, ngg, _scat_row, 0)
        return 0
    lax.fori_loop(0, nrounds, _round, 0)

    @pl.when(w == NW - 1)
    def _():
        def _pv(g, _):
            dst_v[pl.ds(g * L, L)] = jnp.full((L,), EG, jnp.int32)
            src_v[pl.ds(g * L, L)] = jnp.zeros((L,), jnp.int32)
            return 0
        lax.fori_loop(0, PCH // L, _pv, 0)
        pltpu.sync_copy(dst_v, perm_hbm.at[pl.ds(EG, PCH)])
        pltpu.sync_copy(src_v, srcp_hbm.at[pl.ds(EG, PCH)])
        pltpu.sync_copy(src_v, dstp_hbm.at[pl.ds(EG, PCH)])


def _prep_scatter(dst, src, cnt_t, base0, gp, hb, nb):
    _, shift, _, _, _ = gp
    f = pl.kernel(
        functools.partial(_scat_body, shift, nb, hb),
        out_type=[jax.ShapeDtypeStruct((EG + 2 * PCH + 2048,), jnp.int32),
                  jax.ShapeDtypeStruct((EG + 2 * PCH + 2048,), jnp.int32),
                  jax.ShapeDtypeStruct((EG + 2 * PCH + 2048,), jnp.int32),
                  jax.ShapeDtypeStruct((nb,), jnp.int32)],
        mesh=_MESH,
        name="prep_scat",
        scratch_types=[
            pltpu.VMEM((nb * NW,), jnp.int32),
            pltpu.VMEM((PCH,), jnp.int32),
            pltpu.VMEM((PCH,), jnp.int32),
            pltpu.VMEM((nb,), jnp.int32),
            pltpu.VMEM((L,), jnp.int32),
            pltpu.VMEM((PCH // (8 * L), 8 * L), jnp.int32),
            pltpu.VMEM((PCH // (8 * L), 8 * L), jnp.int32),
            pltpu.VMEM((PCH // (8 * L), 8 * L), jnp.int32),
            pltpu.VMEM((PCH // (8 * L), 8 * L), jnp.int32),
            pltpu.SMEM((nb,), jnp.int32),
            pltpu.SemaphoreType.DMA,
        ],
    )
    return f(dst, src, cnt_t.reshape(-1), base0)


def _graph_prep(src, dst, gp):
    _, _, b_pad, _, _ = gp
    zero16 = jnp.zeros((L,), jnp.int32)
    if b_pad <= NB_HALF_MAX:
        cnt = _prep_hist(dst, gp, 0, b_pad)
        perm, srcp, dstp, bst = _prep_scatter(
            dst, src, jnp.transpose(cnt), zero16, gp, 0, b_pad)
        bstart = jnp.concatenate([bst, jnp.full((L,), EG, jnp.int32)])
        return perm, srcp, dstp, bstart
    nb = b_pad // 2
    cnt1 = _prep_hist(dst, gp, 0, nb)
    cnt2 = _prep_hist(dst, gp, nb, nb)
    n1 = jnp.sum(cnt1).astype(jnp.int32)
    base2 = zero16.at[0].set(n1)
    p1, s1, d1, bst1 = _prep_scatter(dst, src, jnp.transpose(cnt1), zero16, gp, 0, nb)
    p2, s2, d2, bst2 = _prep_scatter(dst, src, jnp.transpose(cnt2), base2, gp, nb, nb)
    pos = lax.broadcasted_iota(jnp.int32, p1.shape, 0)
    first = pos < n1
    perm = jnp.where(first, p1, p2)
    srcp = jnp.where(first, s1, s2)
    dstp = jnp.where(first, d1, d2)
    bstart = jnp.concatenate([bst1, bst2, jnp.full((L,), EG, jnp.int32)])
    return perm, srcp, dstp, bstart


def _egc_sc_body(w_bucket, b_pad, bpw, ge_rows,
                 esrc_hbm, edst_hbm, bh_hbm, ge_hbm,
                 perm_hbm, srcp_hbm, dstp_hbm, bst_hbm,
                 m_hbm, acc_hbm,
                 pc, sc_, dc, m_buf, bh_buf, acc, bst16, sema, semb, semc):
    w = _wid()
    zf = jnp.zeros((L,), jnp.float32)

    def _prefetch(sl, cs):
        pltpu.async_copy(perm_hbm.at[pl.ds(cs, KC)], pc.at[sl], semb).wait()
        pltpu.async_copy(srcp_hbm.at[pl.ds(cs, KC)], sc_.at[sl], semb).wait()
        pltpu.async_copy(dstp_hbm.at[pl.ds(cs, KC)], dc.at[sl], semb).wait()
        pltpu.async_copy(ge_hbm.at[pc.at[sl]], m_buf.at[sl], sema)
        pltpu.async_copy(bh_hbm.at[sc_.at[sl]], bh_buf.at[sl], semb)

    def _drain(sl):
        pltpu.make_async_copy(ge_hbm.at[pc.at[sl]], m_buf.at[sl], sema).wait()
        pltpu.make_async_copy(bh_hbm.at[sc_.at[sl]], bh_buf.at[sl], semb).wait()

    def _bucket(bi, _):
        b = w * bpw + bi
        al = (b // 8) * 8
        pltpu.sync_copy(bst_hbm.at[pl.ds(al, L)], bst16)
        bv = bst16[pl.ds(0, L)]
        e0 = _sel_lane(bv, b - al)
        e1 = _sel_lane(bv, b - al + 1)

        def _zr(r, _):
            for j in range(12):
                acc[r, pl.ds(j * L, L)] = zf
            return 0
        lax.fori_loop(0, w_bucket, _zr, 0)

        cs0 = (e0 // 8) * 8
        nch = (e1 - cs0 + KC - 1) // KC

        @pl.when(nch > 0)
        def _():
            _prefetch(0, cs0)

            def _chunk(ci, _):
                sl = lax.rem(ci, 2)
                cs = cs0 + ci * KC
                lo = jnp.maximum(e0 - cs, 0)
                hi = jnp.minimum(e1 - cs, KC)
                pltpu.make_async_copy(ge_hbm.at[pc.at[sl]], m_buf.at[sl],
                                      sema).wait()
                d1 = pltpu.async_copy(esrc_hbm.at[sc_.at[sl]], m_buf.at[sl],
                                      sema, add=True)
                d2 = pltpu.async_copy(edst_hbm.at[dc.at[sl]], m_buf.at[sl],
                                      sema, add=True)
                d1.wait()
                d2.wait()
                pltpu.make_async_copy(bh_hbm.at[sc_.at[sl]], bh_buf.at[sl],
                                      semb).wait()
                pltpu.async_copy(m_buf.at[sl], m_hbm.at[pc.at[sl]], semc).wait()

                _prefetch(1 - sl, cs + KC)

                @plsc.parallel_loop(0, KC, unroll=4)
                def _sig(i):
                    for j in range(6):
                        m = m_buf[sl, i, pl.ds(j * L, L)]
                        sg = 1.0 / (1.0 + jnp.exp(-m))
                        bhv = bh_buf[sl, i, pl.ds(j * L, L)]
                        m_buf[sl, i, pl.ds(j * L, L)] = sg
                        bh_buf[sl, i, pl.ds(j * L, L)] = sg * bhv

                bw = b * w_bucket
                for q in range(KC // L):
                    dls = dc[sl, pl.ds(q * L, L)] - bw
                    for l in range(L):
                        i = q * L + l
                        dl = dls[l]

                        @pl.when((i >= lo) & (i < hi))
                        def _():
                            for j in range(6):
                                plsc.addupdate(acc.at[dl, pl.ds(j * L, L)],
                                               bh_buf[sl, i, pl.ds(j * L, L)])
                                plsc.addupdate(acc.at[dl, pl.ds(96 + j * L, L)],
                                               m_buf[sl, i, pl.ds(j * L, L)])
                return 0
            lax.fori_loop(0, nch, _chunk, 0)
            _drain(lax.rem(nch, 2))
        pltpu.sync_copy(acc, acc_hbm.at[pl.ds(b * w_bucket, w_bucket)])
        return 0
    lax.fori_loop(0, bpw, _bucket, 0)


def _egc_edge_sc(esrc, edst, bh, ge, prep, gp):
    perm, srcp, dstp, bst = prep
    w_bucket, _, b_pad, bpw, s_pad = gp
    f = pl.kernel(
        functools.partial(_egc_sc_body, w_bucket, b_pad, bpw, ge.shape[0]),
        out_type=[jax.ShapeDtypeStruct((EG + PCH, 128), jnp.float32),
                  jax.ShapeDtypeStruct((s_pad, 192), jnp.float32)],
        mesh=_MESH,
        name="egc_edge",
        scratch_types=[
            pltpu.VMEM((2, KC), jnp.int32),
            pltpu.VMEM((2, KC), jnp.int32),
            pltpu.VMEM((2, KC), jnp.int32),
            pltpu.VMEM((2, KC, 128), jnp.float32),
            pltpu.VMEM((2, KC, 128), jnp.float32),
            pltpu.VMEM((w_bucket, 192), jnp.float32),
            pltpu.VMEM((L,), jnp.int32),
            pltpu.SemaphoreType.DMA,
            pltpu.SemaphoreType.DMA,
            pltpu.SemaphoreType.DMA,
        ],
    )
    return f(esrc, edst, bh, ge, perm, srcp, dstp, bst)


def _edge_phase(esrc, edst, bh, ge, prep, gp):
    m_arr, acc = _egc_edge_sc(esrc, edst, bh, ge, prep, gp)
    return m_arr, acc


def _egc_layer(p, tp, prep, gp, x, y, blk_x, blk_y):
    wcat = jnp.concatenate([p["src_gate"]["w"], p["dst_gate"]["w"],
                            p["dst_update"]["w"], p["src_update"]["w"]], axis=1)
    bcat = jnp.concatenate([p["src_gate"]["b"] + tp, p["dst_gate"]["b"],
                            p["dst_update"]["b"], p["src_update"]["b"]])
    esrc, edst, bh, xu = _egc_pre(x, wcat, bcat, blk_x)
    ge = _egc_ge(y, p["edge_gate"]["w"], p["edge_gate"]["b"], blk_y, EG + 8)
    m_arr, acc = _edge_phase(esrc, edst, bh, ge, prep, gp)
    x_new = _egc_post_x(x, xu, acc, p["ln_n"]["g"], p["ln_n"]["b"], blk_x)
    y_new = _egc_post_y(y, m_arr, p["ln_e"]["g"], p["ln_e"]["b"], blk_y)
    return x_new, y_new


def kernel(edge_index, lg_edge_index, atom_feats, bondlength, cos_angles, timesteps, params):
    src, dst = edge_index[0], edge_index[1]
    lsrc, ldst = lg_edge_index[0], lg_edge_index[1]
    n = atom_feats.shape[0]
    e = bondlength.shape[0]

    egc_ps = ([lp["node"] for lp in params["alignn"]]
              + [lp["edge"] for lp in params["alignn"]]
              + list(params["gcn"])
              + [params["edges_l1"], params["edges_l2"], params["atoms_l"]])
    wp_all = jnp.concatenate([q["time_proj"]["w"] for q in egc_ps], axis=1)
    bp_all = jnp.concatenate([q["time_proj"]["b"] for q in egc_ps])
    tp_all = _time_tp(timesteps, params, len(egc_ps), wp_all, bp_all)

    x = _atom_emb(atom_feats, params["atom_emb"])
    y = _emb2(bondlength, params["edge_emb"][0], params["edge_emb"][1], 0.0, 8.0, 80, BLK_E)
    z = _emb2(cos_angles, params["angle_emb"][0], params["angle_emb"][1], -1.0, 1.0, 40, BLK_E)

    prep_n = _graph_prep(src, dst, GP_NODE)
    prep_l = _graph_prep(lsrc, ldst, GP_EDGE)

    na = len(params["alignn"])
    for i, lp in enumerate(params["alignn"]):
        x, m = _egc_layer(lp["node"], tp_all[i], prep_n, GP_NODE, x, y, BLK_N, BLK_E)
        y, z = _egc_layer(lp["edge"], tp_all[na + i], prep_l, GP_EDGE, m, z, BLK_E, BLK_E)
    for j, lp in enumerate(params["gcn"]):
        x, y = _egc_layer(lp, tp_all[2 * na + j], prep_n, GP_NODE, x, y, BLK_N, BLK_E)
    xe, ye = _egc_layer(params["edges_l1"], tp_all[9], prep_n, GP_NODE, x, y, BLK_N, BLK_E)
    xe, ye = _egc_layer(params["edges_l2"], tp_all[10], prep_n, GP_NODE, xe, ye, BLK_N, BLK_E)
    edge_out = _readout(params["edges_ro"], ye, BLK_E)
    xa, ya = _egc_layer(params["atoms_l"], tp_all[11], prep_n, GP_NODE, x, y, BLK_N, BLK_E)
    atom_out = _readout(params["atoms_ro"], xa, BLK_N)
    return jnp.concatenate([atom_out, edge_out], axis=0)


# speedup vs baseline: 1.6286x; 1.0618x over previous
import functools
import math

import jax
import jax.numpy as jnp
from jax import lax
from jax.experimental import pallas as pl
from jax.experimental.pallas import tpu as pltpu, tpu_sc as plsc

HID = 96
EMB = 64

N_NODES = 50000
N_EDGES = 800000
BLK_N = 2000
BLK_E = 3200


def _ln_silu(h, g, b):
    m = h.mean(axis=-1, keepdims=True)
    v = ((h - m) ** 2).mean(axis=-1, keepdims=True)
    h = (h - m) / jnp.sqrt(v + 1e-5) * g + b
    return h * jax.nn.sigmoid(h)


def _emb2_kernel(xs_ref, w1_ref, b1_ref, g1_ref, n1_ref, w2_ref, b2_ref,
                 g2_ref, n2_ref, o_ref, *, vmin, vmax, bins):
    xs = xs_ref[...]
    delta = (vmax - vmin) / (bins - 1)
    centers = vmin + delta * lax.broadcasted_iota(jnp.int32, (1, bins), 1).astype(jnp.float32)
    gamma = 1.0 / (delta * delta)
    r = jnp.exp(-gamma * (xs - centers) ** 2)
    h = _ln_silu(r @ w1_ref[...] + b1_ref[...], g1_ref[...], n1_ref[...])
    h = _ln_silu(h @ w2_ref[...] + b2_ref[...], g2_ref[...], n2_ref[...])
    o_ref[...] = h


def _emb2(xs, p1, p2, vmin, vmax, bins, blk):
    rows = xs.shape[0]
    d1 = p1["lin"]["w"].shape[1]
    d2 = p2["lin"]["w"].shape[1]
    f = pl.pallas_call(
        functools.partial(_emb2_kernel, vmin=vmin, vmax=vmax, bins=bins),
        grid=(rows // blk,),
        in_specs=[
            pl.BlockSpec((blk, 1), lambda i: (i, 0)),
            pl.BlockSpec((bins, d1), lambda i: (0, 0)),
            pl.BlockSpec((1, d1), lambda i: (0, 0)),
            pl.BlockSpec((1, d1), lambda i: (0, 0)),
            pl.BlockSpec((1, d1), lambda i: (0, 0)),
            pl.BlockSpec((d1, d2), lambda i: (0, 0)),
            pl.BlockSpec((1, d2), lambda i: (0, 0)),
            pl.BlockSpec((1, d2), lambda i: (0, 0)),
            pl.BlockSpec((1, d2), lambda i: (0, 0)),
        ],
        out_specs=pl.BlockSpec((blk, d2), lambda i: (i, 0)),
        out_shape=jax.ShapeDtypeStruct((rows, d2), jnp.float32),
    )
    r2 = lambda a: a.reshape(1, -1)
    return f(xs[:, None], p1["lin"]["w"], r2(p1["lin"]["b"]), r2(p1["ln"]["g"]),
             r2(p1["ln"]["b"]), p2["lin"]["w"], r2(p2["lin"]["b"]),
             r2(p2["ln"]["g"]), r2(p2["ln"]["b"]))


def _atom_emb_kernel(x_ref, w_ref, b_ref, g_ref, n_ref, o_ref):
    h = x_ref[...] @ w_ref[...] + b_ref[...]
    o_ref[...] = _ln_silu(h, g_ref[...], n_ref[...])


def _atom_emb(x, p):
    rows, din = x.shape
    f = pl.pallas_call(
        _atom_emb_kernel,
        grid=(rows // BLK_N,),
        in_specs=[
            pl.BlockSpec((BLK_N, din), lambda i: (i, 0)),
            pl.BlockSpec((din, HID), lambda i: (0, 0)),
            pl.BlockSpec((1, HID), lambda i: (0, 0)),
            pl.BlockSpec((1, HID), lambda i: (0, 0)),
            pl.BlockSpec((1, HID), lambda i: (0, 0)),
        ],
        out_specs=pl.BlockSpec((BLK_N, HID), lambda i: (i, 0)),
        out_shape=jax.ShapeDtypeStruct((rows, HID), jnp.float32),
    )
    r2 = lambda a: a.reshape(1, -1)
    return f(x, p["lin"]["w"], r2(p["lin"]["b"]), r2(p["ln"]["g"]), r2(p["ln"]["b"]))


def _time_kernel(ts_ref, w1_ref, b1_ref, g1_ref, n1_ref, w2_ref, b2_ref,
                 g2_ref, n2_ref, wp_ref, bp_ref, o_ref):
    ts = ts_ref[...]
    half = EMB // 2
    fr = math.log(10000.0) / (half - 1)
    freqs = jnp.exp(lax.broadcasted_iota(jnp.int32, (1, half), 1).astype(jnp.float32) * -fr)
    a = ts * freqs
    t = jnp.concatenate([jnp.sin(a), jnp.cos(a)], axis=1)
    t = _ln_silu(t @ w1_ref[...] + b1_ref[...], g1_ref[...], n1_ref[...])
    t = _ln_silu(t @ w2_ref[...] + b2_ref[...], g2_ref[...], n2_ref[...])
    o_ref[...] = t @ wp_ref[...] + bp_ref[...]


def _time_tp(timesteps, params, n_layers_tp, wp_all, bp_all):
    p1, p2 = params["time_emb"]
    ts8 = jnp.zeros((8, 1), jnp.float32).at[0, 0].set(timesteps[0])
    r2 = lambda a: a.reshape(1, -1)
    f = pl.pallas_call(
        _time_kernel,
        out_shape=jax.ShapeDtypeStruct((8, n_layers_tp * HID), jnp.float32),
    )
    out = f(ts8, p1["lin"]["w"], r2(p1["lin"]["b"]), r2(p1["ln"]["g"]), r2(p1["ln"]["b"]),
            p2["lin"]["w"], r2(p2["lin"]["b"]), r2(p2["ln"]["g"]), r2(p2["ln"]["b"]),
            wp_all, r2(bp_all))
    return out[0].reshape(n_layers_tp, HID)


def _pre_kernel(x_ref, w_ref, b_ref, esrc_ref, edst_ref, bh_ref, xu_ref):
    r = x_ref[...] @ w_ref[...] + b_ref[...]
    blk = r.shape[0]
    z = jnp.zeros((blk, 128 - HID), jnp.float32)
    esrc_ref[...] = jnp.concatenate([r[:, 0:96], z], axis=1)
    edst_ref[...] = jnp.concatenate([r[:, 96:192], z], axis=1)
    bh_ref[...] = jnp.concatenate([r[:, 192:288], z], axis=1)
    xu_ref[...] = r[:, 288:384]


def _egc_pre(x, wcat, bcat, blk):
    rows = x.shape[0]
    f = pl.pallas_call(
        _pre_kernel,
        grid=(rows // blk,),
        in_specs=[
            pl.BlockSpec((blk, HID), lambda i: (i, 0)),
            pl.BlockSpec((HID, 384), lambda i: (0, 0)),
            pl.BlockSpec((1, 384), lambda i: (0, 0)),
        ],
        out_specs=[
            pl.BlockSpec((blk, 128), lambda i: (i, 0)),
            pl.BlockSpec((blk, 128), lambda i: (i, 0)),
            pl.BlockSpec((blk, 128), lambda i: (i, 0)),
            pl.BlockSpec((blk, HID), lambda i: (i, 0)),
        ],
        out_shape=[
            jax.ShapeDtypeStruct((rows, 128), jnp.float32),
            jax.ShapeDtypeStruct((rows, 128), jnp.float32),
            jax.ShapeDtypeStruct((rows, 128), jnp.float32),
            jax.ShapeDtypeStruct((rows, HID), jnp.float32),
        ],
    )
    return f(x, wcat, bcat.reshape(1, -1))


def _ge_kernel(y_ref, w_ref, b_ref, o_ref):
    r = y_ref[...] @ w_ref[...] + b_ref[...]
    blk = r.shape[0]
    z = jnp.zeros((blk, 128 - HID), jnp.float32)
    o_ref[...] = jnp.concatenate([r, z], axis=1)


def _egc_ge(y, w, b, blk, out_rows):
    rows = y.shape[0]
    f = pl.pallas_call(
        _ge_kernel,
        grid=(rows // blk,),
        in_specs=[
            pl.BlockSpec((blk, HID), lambda i: (i, 0)),
            pl.BlockSpec((HID, HID), lambda i: (0, 0)),
            pl.BlockSpec((1, HID), lambda i: (0, 0)),
        ],
        out_specs=pl.BlockSpec((blk, 128), lambda i: (i, 0)),
        out_shape=jax.ShapeDtypeStruct((out_rows, 128), jnp.float32),
    )
    return f(y, w, b.reshape(1, -1))


def _post_x_kernel(x_ref, xu_ref, acc_ref, g_ref, b_ref, o_ref):
    acc = acc_ref[...]
    h = acc[:, 0:96] / (acc[:, 96:192] + 1e-6)
    xo = _ln_silu(xu_ref[...] + h, g_ref[...], b_ref[...])
    o_ref[...] = x_ref[...] + xo


def _egc_post_x(x, xu, acc, g, b, blk):
    rows = x.shape[0]
    f = pl.pallas_call(
        _post_x_kernel,
        grid=(rows // blk,),
        in_specs=[
            pl.BlockSpec((blk, HID), lambda i: (i, 0)),
            pl.BlockSpec((blk, HID), lambda i: (i, 0)),
            pl.BlockSpec((blk, 192), lambda i: (i, 0)),
            pl.BlockSpec((1, HID), lambda i: (0, 0)),
            pl.BlockSpec((1, HID), lambda i: (0, 0)),
        ],
        out_specs=pl.BlockSpec((blk, HID), lambda i: (i, 0)),
        out_shape=jax.ShapeDtypeStruct((rows, HID), jnp.float32),
    )
    return f(x, xu, acc, g.reshape(1, -1), b.reshape(1, -1))


def _post_y_kernel(y_ref, m_ref, g_ref, b_ref, o_ref):
    yo = _ln_silu(m_ref[...][:, 0:96], g_ref[...], b_ref[...])
    o_ref[...] = y_ref[...] + yo


def _egc_post_y(y, m_arr, g, b, blk):
    rows = y.shape[0]
    f = pl.pallas_call(
        _post_y_kernel,
        grid=(rows // blk,),
        in_specs=[
            pl.BlockSpec((blk, HID), lambda i: (i, 0)),
            pl.BlockSpec((blk, 128), lambda i: (i, 0)),
            pl.BlockSpec((1, HID), lambda i: (0, 0)),
            pl.BlockSpec((1, HID), lambda i: (0, 0)),
        ],
        out_specs=pl.BlockSpec((blk, HID), lambda i: (i, 0)),
        out_shape=jax.ShapeDtypeStruct((rows, HID), jnp.float32),
    )
    return f(y, m_arr, g.reshape(1, -1), b.reshape(1, -1))


def _readout_kernel(x_ref, w_ref, b_ref, o_ref):
    o_ref[...] = x_ref[...] @ w_ref[...] + b_ref[...]


def _readout(p, x, blk):
    rows = x.shape[0]
    f = pl.pallas_call(
        _readout_kernel,
        grid=(rows // blk,),
        in_specs=[
            pl.BlockSpec((blk, HID), lambda i: (i, 0)),
            pl.BlockSpec((HID, 1), lambda i: (0, 0)),
            pl.BlockSpec((1, 1), lambda i: (0, 0)),
        ],
        out_specs=pl.BlockSpec((blk, 1), lambda i: (i, 0)),
        out_shape=jax.ShapeDtypeStruct((rows, 1), jnp.float32),
    )
    return f(x, p["w"], p["b"].reshape(1, 1))


NC, NS, L = 2, 16, 16
NW = NC * NS
EG = N_EDGES
PCH = 2048
NCHUNKS = (EG + PCH - 1) // PCH
LAST_N = EG - (NCHUNKS - 1) * PCH
KC = 96

GP_NODE = (128, 7, 416, 13, 416 * 128)
GP_EDGE = (256, 8, 3136, 98, 3136 * 256)
NB_HALF_MAX = 1568
TRASH = EG + PCH

_MESH = plsc.VectorSubcoreMesh(core_axis_name="c", subcore_axis_name="s")


def _wid():
    return lax.axis_index("s") * NC + lax.axis_index("c")


def _lane_iota():
    return lax.iota(jnp.int32, L)


def _sel_lane(vec, k):
    sc = vec[0]
    for l in range(1, L):
        sc = jnp.where(k == l, vec[l], sc)
    return sc


def _hist_body(shift, nb, hb, dst_hbm, cnt_hbm, dst_v, cnt_v, hist_s):
    w = _wid()

    def _z(i, _):
        hist_s[i] = 0
        return 0
    lax.fori_loop(0, nb, _z, 0)

    nrounds = (NCHUNKS - w + NW - 1) // NW

    def _round(k, _):
        c = w + k * NW
        cs = c * PCH

        @pl.when(c < NCHUNKS - 1)
        def _():
            pltpu.sync_copy(dst_hbm.at[pl.ds(cs, PCH)], dst_v)

        @pl.when(c == NCHUNKS - 1)
        def _():
            pltpu.sync_copy(dst_hbm.at[pl.ds(cs, LAST_N)], dst_v.at[pl.ds(0, LAST_N)])

        ng = jnp.where(c == NCHUNKS - 1, LAST_N // L, PCH // L)

        def _grp(g, _):
            b16 = lax.shift_right_logical(dst_v[pl.ds(g * L, L)], shift) - hb
            for l in range(L):
                b = b16[l]

                @pl.when((b >= 0) & (b < nb))
                def _():
                    hist_s[b] = hist_s[b] + 1
            return 0
        lax.fori_loop(0, ng, _grp, 0)
        return 0
    lax.fori_loop(0, nrounds, _round, 0)

    def _flush(g, _):
        v = jnp.zeros((L,), jnp.int32)
        io = _lane_iota()
        for l in range(L):
            v = jnp.where(io == l, hist_s[g * L + l], v)
        cnt_v[pl.ds(g * L, L)] = v
        return 0
    lax.fori_loop(0, nb // L, _flush, 0)
    pltpu.sync_copy(cnt_v, cnt_hbm.at[w])


def _prep_hist(dst, gp, hb, nb):
    _, shift, _, _, _ = gp
    f = pl.kernel(
        functools.partial(_hist_body, shift, nb, hb),
        out_type=[jax.ShapeDtypeStruct((NW, nb), jnp.int32)],
        mesh=_MESH,
        name="prep_hist",
        scratch_types=[
            pltpu.VMEM((PCH,), jnp.int32),
            pltpu.VMEM((nb,), jnp.int32),
            pltpu.SMEM((nb,), jnp.int32),
        ],
    )
    return f(dst)[0]


def _scat_body(shift, nb, hb, dst_hbm, src_hbm, cnt_hbm, base_hbm,
               perm_hbm, srcp_hbm, dstp_hbm, bst_hbm,
               cnt_v, dst_v, src_v, bst_v, base_v,
               pos_b, id_b, src_b, dst_b, off_s, sem):
    w = _wid()
    io = _lane_iota()
    pltpu.sync_copy(cnt_hbm, cnt_v)
    pltpu.sync_copy(base_hbm, base_v)
    base0 = base_v[pl.ds(0, L)][0]

    def _off(b, base):
        cv0 = cnt_v[pl.ds(b * NW, L)]
        cv1 = cnt_v[pl.ds(b * NW + L, L)]
        excl = jnp.int32(0)
        tot = jnp.int32(0)
        for l in range(L):
            el = cv0[l]
            excl = excl + jnp.where(w > l, el, 0)
            tot = tot + el
        for l in range(L):
            el = cv1[l]
            excl = excl + jnp.where(w > L + l, el, 0)
            tot = tot + el
        off_s[b] = base + excl
        return base + tot
    lax.fori_loop(0, nb, _off, base0)

    @pl.when(w == 0)
    def _():
        def _fl(g, _):
            v = jnp.zeros((L,), jnp.int32)
            for l in range(L):
                v = jnp.where(io == l, off_s[g * L + l], v)
            bst_v[pl.ds(g * L, L)] = v
            return 0
        lax.fori_loop(0, nb // L, _fl, 0)
        pltpu.sync_copy(bst_v, bst_hbm)

    nrounds = (NCHUNKS - w + NW - 1) // NW

    def _round(k, _):
        c = w + k * NW
        cs = c * PCH

        @pl.when(c < NCHUNKS - 1)
        def _():
            pltpu.sync_copy(dst_hbm.at[pl.ds(cs, PCH)], dst_v)
            pltpu.sync_copy(src_hbm.at[pl.ds(cs, PCH)], src_v)

        @pl.when(c == NCHUNKS - 1)
        def _():
            pltpu.sync_copy(dst_hbm.at[pl.ds(cs, LAST_N)], dst_v.at[pl.ds(0, LAST_N)])
            pltpu.sync_copy(src_hbm.at[pl.ds(cs, LAST_N)], src_v.at[pl.ds(0, LAST_N)])

        ngg = jnp.where(c == NCHUNKS - 1, LAST_N // (8 * L), PCH // (8 * L))

        def _row(gg, _):
            for q in range(8):
                o = gg * 8 * L + q * L
                d16 = dst_v[pl.ds(o, L)]
                s16 = src_v[pl.ds(o, L)]
                b16 = lax.shift_right_logical(d16, shift) - hb
                id16 = io + (cs + o)
                posv = TRASH + o + io
                for l in range(L):
                    b = b16[l]
                    inh = (b >= 0) & (b < nb)
                    bc = jnp.clip(b, 0, nb - 1)

                    @pl.when(inh)
                    def _():
                        off_s[bc] = off_s[bc] + 1

                    p2 = jnp.where(inh, off_s[bc] - 1, TRASH + o + l)
                    posv = jnp.where(io == l, p2, posv)
                pos_b[gg, pl.ds(q * L, L)] = posv
                id_b[gg, pl.ds(q * L, L)] = id16
                src_b[gg, pl.ds(q * L, L)] = s16
                dst_b[gg, pl.ds(q * L, L)] = d16
            return 0
        lax.fori_loop(0, ngg, _row, 0)

        def _scat_row(j, _):
            c1 = pltpu.async_copy(id_b.at[j], perm_hbm.at[pos_b.at[j]], sem)
            c2 = pltpu.async_copy(src_b.at[j], srcp_hbm.at[pos_b.at[j]], sem)
            c3 = pltpu.async_copy(dst_b.at[j], dstp_hbm.at[pos_b.at[j]], sem)
            c1.wait()
            c2.wait()
            c3.wait()
            return 0
        lax.fori_loop(0, ngg, _scat_row, 0)
        return 0
    lax.fori_loop(0, nrounds, _round, 0)

    @pl.when(w == NW - 1)
    def _():
        def _pv(g, _):
            dst_v[pl.ds(g * L, L)] = jnp.full((L,), EG, jnp.int32)
            src_v[pl.ds(g * L, L)] = jnp.zeros((L,), jnp.int32)
            return 0
        lax.fori_loop(0, PCH // L, _pv, 0)
        pltpu.sync_copy(dst_v, perm_hbm.at[pl.ds(EG, PCH)])
        pltpu.sync_copy(src_v, srcp_hbm.at[pl.ds(EG, PCH)])
        pltpu.sync_copy(src_v, dstp_hbm.at[pl.ds(EG, PCH)])


def _prep_scatter(dst, src, cnt_t, base0, gp, hb, nb):
    _, shift, _, _, _ = gp
    f = pl.kernel(
        functools.partial(_scat_body, shift, nb, hb),
        out_type=[jax.ShapeDtypeStruct((EG + 2 * PCH + 2048,), jnp.int32),
                  jax.ShapeDtypeStruct((EG + 2 * PCH + 2048,), jnp.int32),
                  jax.ShapeDtypeStruct((EG + 2 * PCH + 2048,), jnp.int32),
                  jax.ShapeDtypeStruct((nb,), jnp.int32)],
        mesh=_MESH,
        name="prep_scat",
        scratch_types=[
            pltpu.VMEM((nb * NW,), jnp.int32),
            pltpu.VMEM((PCH,), jnp.int32),
            pltpu.VMEM((PCH,), jnp.int32),
            pltpu.VMEM((nb,), jnp.int32),
            pltpu.VMEM((L,), jnp.int32),
            pltpu.VMEM((PCH // (8 * L), 8 * L), jnp.int32),
            pltpu.VMEM((PCH // (8 * L), 8 * L), jnp.int32),
            pltpu.VMEM((PCH // (8 * L), 8 * L), jnp.int32),
            pltpu.VMEM((PCH // (8 * L), 8 * L), jnp.int32),
            pltpu.SMEM((nb,), jnp.int32),
            pltpu.SemaphoreType.DMA,
        ],
    )
    return f(dst, src, cnt_t.reshape(-1), base0)


def _graph_prep(src, dst, gp):
    _, _, b_pad, _, _ = gp
    zero16 = jnp.zeros((L,), jnp.int32)
    if b_pad <= NB_HALF_MAX:
        cnt = _prep_hist(dst, gp, 0, b_pad)
        perm, srcp, dstp, bst = _prep_scatter(
            dst, src, jnp.transpose(cnt), zero16, gp, 0, b_pad)
        bstart = jnp.concatenate([bst, jnp.full((L,), EG, jnp.int32)])
        return perm, srcp, dstp, bstart
    nb = b_pad // 2
    cnt1 = _prep_hist(dst, gp, 0, nb)
    cnt2 = _prep_hist(dst, gp, nb, nb)
    n1 = jnp.sum(cnt1).astype(jnp.int32)
    base2 = zero16.at[0].set(n1)
    p1, s1, d1, bst1 = _prep_scatter(dst, src, jnp.transpose(cnt1), zero16, gp, 0, nb)
    p2, s2, d2, bst2 = _prep_scatter(dst, src, jnp.transpose(cnt2), base2, gp, nb, nb)
    pos = lax.broadcasted_iota(jnp.int32, p1.shape, 0)
    first = pos < n1
    perm = jnp.where(first, p1, p2)
    srcp = jnp.where(first, s1, s2)
    dstp = jnp.where(first, d1, d2)
    bstart = jnp.concatenate([bst1, bst2, jnp.full((L,), EG, jnp.int32)])
    return perm, srcp, dstp, bstart


def _egc_sc_body(w_bucket, b_pad, bpw, ge_rows,
                 esrc_hbm, edst_hbm, bh_hbm, ge_hbm,
                 perm_hbm, srcp_hbm, dstp_hbm, bst_hbm,
                 m_hbm, acc_hbm,
                 pc, sc_, dc, m_buf, bh_buf, s_buf, acc, bst16, sema, semb, semc):
    w = _wid()
    zf = jnp.zeros((L,), jnp.float32)

    def _prefetch(sl, cs):
        a1 = pltpu.async_copy(perm_hbm.at[pl.ds(cs, KC)], pc.at[sl], semb)
        a2 = pltpu.async_copy(srcp_hbm.at[pl.ds(cs, KC)], sc_.at[sl], semb)
        a3 = pltpu.async_copy(dstp_hbm.at[pl.ds(cs, KC)], dc.at[sl], semb)
        a1.wait()
        a2.wait()
        a3.wait()
        pltpu.async_copy(ge_hbm.at[pc.at[sl]], m_buf.at[sl], sema)
        pltpu.async_copy(bh_hbm.at[sc_.at[sl]], bh_buf.at[sl], semb)

    def _drain(sl):
        pltpu.make_async_copy(ge_hbm.at[pc.at[sl]], m_buf.at[sl], sema).wait()
        pltpu.make_async_copy(bh_hbm.at[sc_.at[sl]], bh_buf.at[sl], semb).wait()

    def _bucket(bi, _):
        b = w * bpw + bi
        al = (b // 8) * 8
        pltpu.sync_copy(bst_hbm.at[pl.ds(al, L)], bst16)
        bv = bst16[pl.ds(0, L)]
        e0 = _sel_lane(bv, b - al)
        e1 = _sel_lane(bv, b - al + 1)

        def _zr(r, _):
            for j in range(12):
                acc[r, pl.ds(j * L, L)] = zf
            return 0
        lax.fori_loop(0, w_bucket, _zr, 0)

        cs0 = (e0 // 8) * 8
        nch = (e1 - cs0 + KC - 1) // KC

        @pl.when(nch > 0)
        def _():
            _prefetch(0, cs0)

            def _chunk(ci, _):
                sl = lax.rem(ci, 2)
                cs = cs0 + ci * KC
                lo = jnp.maximum(e0 - cs, 0)
                hi = jnp.minimum(e1 - cs, KC)
                pltpu.make_async_copy(ge_hbm.at[pc.at[sl]], m_buf.at[sl],
                                      sema).wait()
                d1 = pltpu.async_copy(esrc_hbm.at[sc_.at[sl]], m_buf.at[sl],
                                      sema, add=True)
                d2 = pltpu.async_copy(edst_hbm.at[dc.at[sl]], m_buf.at[sl],
                                      sema, add=True)
                d1.wait()
                d2.wait()
                pltpu.make_async_copy(bh_hbm.at[sc_.at[sl]], bh_buf.at[sl],
                                      semb).wait()
                pltpu.async_copy(m_buf.at[sl], m_hbm.at[pc.at[sl]], semc)

                @pl.when(ci > 0)
                def _():
                    pltpu.make_async_copy(m_buf.at[1 - sl],
                                          m_hbm.at[pc.at[1 - sl]], semc).wait()

                _prefetch(1 - sl, cs + KC)

                @plsc.parallel_loop(0, KC, unroll=4)
                def _sig(i):
                    for j in range(6):
                        m = m_buf[sl, i, pl.ds(j * L, L)]
                        sg = 1.0 / (1.0 + jnp.exp(-m))
                        bhv = bh_buf[sl, i, pl.ds(j * L, L)]
                        s_buf[i, pl.ds(j * L, L)] = sg
                        bh_buf[sl, i, pl.ds(j * L, L)] = sg * bhv

                bw = b * w_bucket
                for q in range(KC // L):
                    dls = dc[sl, pl.ds(q * L, L)] - bw
                    for l in range(L):
                        i = q * L + l
                        dl = dls[l]

                        @pl.when((i >= lo) & (i < hi))
                        def _():
                            for j in range(6):
                                plsc.addupdate(acc.at[dl, pl.ds(j * L, L)],
                                               bh_buf[sl, i, pl.ds(j * L, L)])
                                plsc.addupdate(acc.at[dl, pl.ds(96 + j * L, L)],
                                               s_buf[i, pl.ds(j * L, L)])
                return 0
            lax.fori_loop(0, nch, _chunk, 0)
            sll = lax.rem(nch - 1, 2)
            pltpu.make_async_copy(m_buf.at[sll], m_hbm.at[pc.at[sll]],
                                  semc).wait()
            _drain(lax.rem(nch, 2))
        pltpu.sync_copy(acc, acc_hbm.at[pl.ds(b * w_bucket, w_bucket)])
        return 0
    lax.fori_loop(0, bpw, _bucket, 0)


def _egc_edge_sc(esrc, edst, bh, ge, prep, gp):
    perm, srcp, dstp, bst = prep
    w_bucket, _, b_pad, bpw, s_pad = gp
    f = pl.kernel(
        functools.partial(_egc_sc_body, w_bucket, b_pad, bpw, ge.shape[0]),
        out_type=[jax.ShapeDtypeStruct((EG + PCH, 128), jnp.float32),
                  jax.ShapeDtypeStruct((s_pad, 192), jnp.float32)],
        mesh=_MESH,
        name="egc_edge",
        scratch_types=[
            pltpu.VMEM((2, KC), jnp.int32),
            pltpu.VMEM((2, KC), jnp.int32),
            pltpu.VMEM((2, KC), jnp.int32),
            pltpu.VMEM((2, KC, 128), jnp.float32),
            pltpu.VMEM((2, KC, 128), jnp.float32),
            pltpu.VMEM((KC, 128), jnp.float32),
            pltpu.VMEM((w_bucket, 192), jnp.float32),
            pltpu.VMEM((L,), jnp.int32),
            pltpu.SemaphoreType.DMA,
            pltpu.SemaphoreType.DMA,
            pltpu.SemaphoreType.DMA,
        ],
    )
    return f(esrc, edst, bh, ge, perm, srcp, dstp, bst)


def _edge_phase(esrc, edst, bh, ge, prep, gp):
    m_arr, acc = _egc_edge_sc(esrc, edst, bh, ge, prep, gp)
    return m_arr, acc


def _egc_layer(p, tp, prep, gp, x, y, blk_x, blk_y):
    wcat = jnp.concatenate([p["src_gate"]["w"], p["dst_gate"]["w"],
                            p["dst_update"]["w"], p["src_update"]["w"]], axis=1)
    bcat = jnp.concatenate([p["src_gate"]["b"] + tp, p["dst_gate"]["b"],
                            p["dst_update"]["b"], p["src_update"]["b"]])
    esrc, edst, bh, xu = _egc_pre(x, wcat, bcat, blk_x)
    ge = _egc_ge(y, p["edge_gate"]["w"], p["edge_gate"]["b"], blk_y, EG + 8)
    m_arr, acc = _edge_phase(esrc, edst, bh, ge, prep, gp)
    x_new = _egc_post_x(x, xu, acc, p["ln_n"]["g"], p["ln_n"]["b"], blk_x)
    y_new = _egc_post_y(y, m_arr, p["ln_e"]["g"], p["ln_e"]["b"], blk_y)
    return x_new, y_new


def kernel(edge_index, lg_edge_index, atom_feats, bondlength, cos_angles, timesteps, params):
    src, dst = edge_index[0], edge_index[1]
    lsrc, ldst = lg_edge_index[0], lg_edge_index[1]
    n = atom_feats.shape[0]
    e = bondlength.shape[0]

    egc_ps = ([lp["node"] for lp in params["alignn"]]
              + [lp["edge"] for lp in params["alignn"]]
              + list(params["gcn"])
              + [params["edges_l1"], params["edges_l2"], params["atoms_l"]])
    wp_all = jnp.concatenate([q["time_proj"]["w"] for q in egc_ps], axis=1)
    bp_all = jnp.concatenate([q["time_proj"]["b"] for q in egc_ps])
    tp_all = _time_tp(timesteps, params, len(egc_ps), wp_all, bp_all)

    x = _atom_emb(atom_feats, params["atom_emb"])
    y = _emb2(bondlength, params["edge_emb"][0], params["edge_emb"][1], 0.0, 8.0, 80, BLK_E)
    z = _emb2(cos_angles, params["angle_emb"][0], params["angle_emb"][1], -1.0, 1.0, 40, BLK_E)

    prep_n = _graph_prep(src, dst, GP_NODE)
    prep_l = _graph_prep(lsrc, ldst, GP_EDGE)

    na = len(params["alignn"])
    for i, lp in enumerate(params["alignn"]):
        x, m = _egc_layer(lp["node"], tp_all[i], prep_n, GP_NODE, x, y, BLK_N, BLK_E)
        y, z = _egc_layer(lp["edge"], tp_all[na + i], prep_l, GP_EDGE, m, z, BLK_E, BLK_E)
    for j, lp in enumerate(params["gcn"]):
        x, y = _egc_layer(lp, tp_all[2 * na + j], prep_n, GP_NODE, x, y, BLK_N, BLK_E)
    xe, ye = _egc_layer(params["edges_l1"], tp_all[9], prep_n, GP_NODE, x, y, BLK_N, BLK_E)
    xe, ye = _egc_layer(params["edges_l2"], tp_all[10], prep_n, GP_NODE, xe, ye, BLK_N, BLK_E)
    edge_out = _readout(params["edges_ro"], ye, BLK_E)
    xa, ya = _egc_layer(params["atoms_l"], tp_all[11], prep_n, GP_NODE, x, y, BLK_N, BLK_E)
    atom_out = _readout(params["atoms_ro"], xa, BLK_N)
    return jnp.concatenate([atom_out, edge_out], axis=0)
